# Initial kernel scaffold; baseline (speedup 1.0000x reference)
import jax
import jax.numpy as jnp
from jax.experimental import pallas as pl


def kernel(x, x_clique, graph_lpe, edge_index_graph, edge_attr_graph, atom2clique_row, atom2clique_col, atom_emb, clique_emb, clique_W, clique_b, bond_emb, eps, W1, b1, bn1_g, bn1_b, W2, b2, gn_g, gn_b, sn_g, sn_b, a2c_W, a2c_b, c2a_W, c2a_b):
    raise NotImplementedError("write your pallas kernel here")


# speedup vs baseline: 2.9809x; 2.9809x over previous
import functools

import jax
import jax.numpy as jnp
from jax import lax
from jax.experimental import pallas as pl
from jax.experimental.pallas import tpu as pltpu
from jax.experimental.pallas import tpu_sc as plsc

N = 50000
NC = 25000
E = 800000
A = 50000
H = 64

NCORE = 2
NSUB = 16
CH = 128

E_PAD = 800768
A_PAD = 51200
N_PAD = 50048
NC_PAD = 25088

HALF_N = N // 2
HALF_NC = NC // 2
SP_N = 25088
SP_NC = 12544

_MESH = plsc.VectorSubcoreMesh(core_axis_name="c", subcore_axis_name="s")
_SC_PARAMS = pltpu.CompilerParams(use_tc_tiling_on_sc=False)


def _relu_add_inplace(abuf, bbuf, rows):
    @pl.loop(0, rows, unroll=8)
    def _(r):
        for q in range(H // 16):
            sl = pl.ds(q * 16, 16)
            abuf[r, sl] = jnp.maximum(abuf[r, sl] + bbuf[r, sl], 0.0)


def _local_idx(dv, lo, half):
    valid = (dv >= lo) & (dv < lo + half)
    return jnp.where(valid, dv - lo, -1)


@functools.partial(
    pl.kernel,
    out_type=(
        jax.ShapeDtypeStruct((N_PAD, H), jnp.float32),
        jax.ShapeDtypeStruct((NC_PAD, H), jnp.float32),
    ),
    mesh=_MESH,
    compiler_params=_SC_PARAMS,
    scratch_types=[
        pltpu.VMEM((CH,), jnp.int32),
        pltpu.VMEM((CH, H), jnp.float32),
        pltpu.VMEM((CH, H), jnp.float32),
    ],
)
def _encode_sc(t9_hbm, idx9_hbm, t4_hbm, xc_hbm, h0_out, hc0_out,
               idxv, acc, tmp):
    w = lax.axis_index("s") * NCORE + lax.axis_index("c")

    n_chunks = N_PAD // CH

    @pl.loop(0, 13)
    def _(jj):
        j = w * 13 + jj

        @pl.when(j < n_chunks)
        def _():
            b = j * CH
            pltpu.sync_copy(idx9_hbm.at[pl.ds(b, CH)], idxv)
            pltpu.sync_copy(t9_hbm.at[idxv], acc)
            for i in range(1, 9):
                pltpu.sync_copy(idx9_hbm.at[pl.ds(i * N_PAD + b, CH)], idxv)
                pltpu.sync_copy(t9_hbm.at[idxv], tmp)

                @pl.loop(0, CH, unroll=8)
                def _(r):
                    for q in range(H // 16):
                        sl = pl.ds(q * 16, 16)
                        acc[r, sl] = acc[r, sl] + tmp[r, sl]
            pltpu.sync_copy(acc, h0_out.at[pl.ds(b, CH), :])

    c_chunks = NC_PAD // CH

    @pl.loop(0, 7)
    def _(jj):
        j = w * 7 + jj

        @pl.when(j < c_chunks)
        def _():
            b = j * CH
            pltpu.sync_copy(xc_hbm.at[pl.ds(b, CH)], idxv)
            pltpu.sync_copy(t4_hbm.at[idxv], acc)
            pltpu.sync_copy(acc, hc0_out.at[pl.ds(b, CH), :])


@functools.partial(
    pl.kernel,
    out_type=(
        jax.ShapeDtypeStruct((N, 16), jnp.float32),
        jax.ShapeDtypeStruct((NC, 16), jnp.float32),
    ),
    mesh=_MESH,
    compiler_params=_SC_PARAMS,
    scratch_types=[
        pltpu.VMEM((CH,), jnp.int32),
        pltpu.VMEM((CH, 16), jnp.float32),
        pltpu.VMEM_SHARED((N_PAD, 16), jnp.float32),
    ],
)
def _counts_sc(row_s_hbm, col_s_hbm, ones_hbm, zc_hbm, cnt_a_out, cnt_c_out,
               idxv, ones_v, cnt_sh):
    c = lax.axis_index("c")
    s = lax.axis_index("s")
    pltpu.sync_copy(ones_hbm, ones_v)

    @pl.when(c == 0)
    def _():
        pltpu.sync_copy(zc_hbm, cnt_sh.at[pl.ds(s * 3128, 3128), :])

    @pl.when(c == 1)
    def _():
        pltpu.sync_copy(zc_hbm.at[pl.ds(0, 1564), :],
                        cnt_sh.at[pl.ds(s * 1564, 1564), :])

    plsc.subcore_barrier()

    n_chunks = A_PAD // (NSUB * CH)

    @pl.loop(0, n_chunks)
    def _(j):
        b = s * (A_PAD // NSUB) + j * CH

        @pl.when(c == 0)
        def _():
            pltpu.sync_copy(row_s_hbm.at[pl.ds(b, CH)], idxv)
            pltpu.sync_copy(
                ones_v, cnt_sh.at[plsc.Indices(idxv, ignored_value=-1)],
                add=True)

        @pl.when(c == 1)
        def _():
            pltpu.sync_copy(col_s_hbm.at[pl.ds(b, CH)], idxv)
            pltpu.sync_copy(
                ones_v, cnt_sh.at[plsc.Indices(idxv, ignored_value=-1)],
                add=True)

    plsc.subcore_barrier()

    @pl.when(c == 0)
    def _():
        @pl.when(s < 15)
        def _():
            pltpu.sync_copy(cnt_sh.at[pl.ds(s * 3128, 3128), :],
                            cnt_a_out.at[pl.ds(s * 3128, 3128), :])

        @pl.when(s == 15)
        def _():
            pltpu.sync_copy(cnt_sh.at[pl.ds(15 * 3128, 3080), :],
                            cnt_a_out.at[pl.ds(15 * 3128, 3080), :])

    @pl.when(c == 1)
    def _():
        @pl.when(s < 15)
        def _():
            pltpu.sync_copy(cnt_sh.at[pl.ds(s * 1564, 1564), :],
                            cnt_c_out.at[pl.ds(s * 1564, 1564), :])

        @pl.when(s == 15)
        def _():
            pltpu.sync_copy(cnt_sh.at[pl.ds(15 * 1564, 1540), :],
                            cnt_c_out.at[pl.ds(15 * 1564, 1540), :])


@functools.partial(
    pl.kernel,
    out_type=jax.ShapeDtypeStruct((N, H), jnp.float32),
    mesh=_MESH,
    compiler_params=_SC_PARAMS,
    scratch_types=[
        pltpu.VMEM((CH,), jnp.int32),
        pltpu.VMEM((CH,), jnp.int32),
        pltpu.VMEM((CH,), jnp.int32),
        pltpu.VMEM((CH,), jnp.int32),
        pltpu.VMEM((CH, H), jnp.float32),
        pltpu.VMEM((CH, H), jnp.float32),
        pltpu.VMEM_SHARED((SP_N, H), jnp.float32),
    ],
)
def _edge_sc(h_hbm, bt_hbm, src_hbm, ce_hbm, dst_hbm, z_hbm, aggr_out,
             srcv, cev, dstv, dlv, hbuf, ebuf, aggr_sh):
    c = lax.axis_index("c")
    s = lax.axis_index("s")
    lo = c * HALF_N

    pltpu.sync_copy(z_hbm.at[pl.ds(0, 1568), :],
                    aggr_sh.at[pl.ds(s * 1568, 1568), :])
    plsc.subcore_barrier()

    per_sub = E_PAD // NSUB
    n_chunks = per_sub // CH

    @pl.loop(0, n_chunks)
    def _(j):
        b = s * per_sub + j * CH
        pltpu.sync_copy(src_hbm.at[pl.ds(b, CH)], srcv)
        pltpu.sync_copy(ce_hbm.at[pl.ds(b, CH)], cev)
        pltpu.sync_copy(dst_hbm.at[pl.ds(b, CH)], dstv)
        pltpu.sync_copy(h_hbm.at[srcv], hbuf)
        pltpu.sync_copy(bt_hbm.at[cev], ebuf)
        _relu_add_inplace(hbuf, ebuf, CH)
        for k in range(CH // 16):
            sl = pl.ds(k * 16, 16)
            dlv[sl] = _local_idx(dstv[sl], lo, HALF_N)
        pltpu.sync_copy(
            hbuf, aggr_sh.at[plsc.Indices(dlv, ignored_value=-1)], add=True)

    plsc.subcore_barrier()

    @pl.when(s < 15)
    def _():
        pltpu.sync_copy(aggr_sh.at[pl.ds(s * 1568, 1568), :],
                        aggr_out.at[pl.ds(lo + s * 1568, 1568), :])

    @pl.when(s == 15)
    def _():
        pltpu.sync_copy(aggr_sh.at[pl.ds(15 * 1568, 1480), :],
                        aggr_out.at[pl.ds(lo + 15 * 1568, 1480), :])


def _make_gss(table_rows, out_rows, sp_rows):
    half = out_rows // 2
    span = sp_rows // NSUB
    last = half - 15 * span
    n_chunks = A_PAD // (NSUB * CH)

    @functools.partial(
        pl.kernel,
        out_type=jax.ShapeDtypeStruct((out_rows, H), jnp.float32),
        mesh=_MESH,
        compiler_params=_SC_PARAMS,
        scratch_types=[
            pltpu.VMEM((CH,), jnp.int32),
            pltpu.VMEM((CH,), jnp.int32),
            pltpu.VMEM((CH,), jnp.int32),
            pltpu.VMEM((CH, H), jnp.float32),
            pltpu.VMEM_SHARED((sp_rows, H), jnp.float32),
        ],
    )
    def gss(table_hbm, gidx_hbm, sidx_hbm, z_hbm, out_hbm,
            gv, sv, dlv, buf, acc_sh):
        c = lax.axis_index("c")
        s = lax.axis_index("s")
        lo = c * half

        pltpu.sync_copy(z_hbm.at[pl.ds(0, span), :],
                        acc_sh.at[pl.ds(s * span, span), :])
        plsc.subcore_barrier()

        @pl.loop(0, n_chunks)
        def _(j):
            b = s * (A_PAD // NSUB) + j * CH
            pltpu.sync_copy(gidx_hbm.at[pl.ds(b, CH)], gv)
            pltpu.sync_copy(sidx_hbm.at[pl.ds(b, CH)], sv)
            pltpu.sync_copy(table_hbm.at[gv], buf)
            for k in range(CH // 16):
                sl = pl.ds(k * 16, 16)
                dlv[sl] = _local_idx(sv[sl], lo, half)
            pltpu.sync_copy(
                buf, acc_sh.at[plsc.Indices(dlv, ignored_value=-1)], add=True)

        plsc.subcore_barrier()

        @pl.when(s < 15)
        def _():
            pltpu.sync_copy(acc_sh.at[pl.ds(s * span, span), :],
                            out_hbm.at[pl.ds(lo + s * span, span), :])

        @pl.when(s == 15)
        def _():
            pltpu.sync_copy(acc_sh.at[pl.ds(15 * span, last), :],
                            out_hbm.at[pl.ds(lo + 15 * span, last), :])

    return gss


_a2c_sc = _make_gss(N, NC, SP_NC)
_c2a_sc = _make_gss(NC, N, SP_N)


_BN_EPS = 1e-5
_F32 = jnp.float32


def _dot(a, b):
    return jnp.dot(a, b, preferred_element_type=_F32)


def _stats_update(sums_ref, z, i):
    @pl.when(i == 0)
    def _():
        sums_ref[...] = jnp.zeros_like(sums_ref)

    sums_ref[0:1, :] += jnp.sum(z, axis=0, keepdims=True)
    sums_ref[1:2, :] += jnp.sum(z * z, axis=0, keepdims=True)


def _bn_apply(z, sums, nrows, g, b):
    m = sums[0:1, :] / nrows
    var = sums[1:2, :] / nrows - m * m
    return (z - m) * lax.rsqrt(var + _BN_EPS) * g + b


def _tk1_body(h_ref, aggr_ref, eps_ref, w1_ref, b1_ref, z1_ref, sums_ref):
    i = pl.program_id(0)
    u = (1.0 + eps_ref[0, 0]) * h_ref[...] + aggr_ref[...]
    z = _dot(u, w1_ref[...]) + b1_ref[...]
    z1_ref[...] = z
    _stats_update(sums_ref, z, i)


def _tk2_body(z1_ref, sums1_ref, g1_ref, bb1_ref, w2_ref, b2_ref,
              z2_ref, sums_ref):
    i = pl.program_id(0)
    v = jax.nn.relu(_bn_apply(z1_ref[...], sums1_ref[...], float(N),
                              g1_ref[...], bb1_ref[...]))
    z = _dot(v, w2_ref[...]) + b2_ref[...]
    z2_ref[...] = z
    _stats_update(sums_ref, z, i)


def _tk3_body(z2_ref, sums2_ref, g_ref, b_ref, h_ref):
    h_ref[...] = jax.nn.relu(_bn_apply(z2_ref[...], sums2_ref[...], float(N),
                                       g_ref[...], b_ref[...]))


def _tk4_body(cm_ref, cnt_ref, hc_ref, w_ref, b_ref, out_ref, sums_ref):
    i = pl.program_id(0)
    cm = cm_ref[...] / jnp.maximum(cnt_ref[:, 0:1], 1.0)
    z = hc_ref[...] + jax.nn.relu(_dot(cm, w_ref[...]) + b_ref[...])
    out_ref[...] = z
    _stats_update(sums_ref, z, i)


def _tk5_body(zp_ref, sums_ref, g_ref, b_ref, out_ref):
    out_ref[...] = jax.nn.relu(_bn_apply(zp_ref[...], sums_ref[...], float(NC),
                                         g_ref[...], b_ref[...]))


def _tk6_body(am_ref, cnt_ref, h_ref, w_ref, b_ref, out_ref):
    am = am_ref[...] / jnp.maximum(cnt_ref[:, 0:1], 1.0)
    out_ref[...] = h_ref[...] + jax.nn.relu(_dot(am, w_ref[...]) + b_ref[...])


def _row_spec(bs, cols):
    return pl.BlockSpec((bs, cols), lambda i: (i, 0))


def _full_spec(shape):
    return pl.BlockSpec(shape, lambda i: tuple(0 for _ in shape))


_BN_ROWS = 2000
_BC_ROWS = 1000


def _tc_call(body, grid, in_specs, out_specs, out_shapes):
    return pl.pallas_call(
        body, grid=(grid,), in_specs=in_specs, out_specs=out_specs,
        out_shape=out_shapes)


def _tc_call1(*args):
    def run(*ins):
        (out,) = _tc_call(*args)(*ins)
        return out
    return run


def _run_tk1(h, aggr, eps_l, w1, b1):
    return _tc_call(
        _tk1_body, N // _BN_ROWS,
        [_row_spec(_BN_ROWS, H), _row_spec(_BN_ROWS, H), _full_spec((8, 128)),
         _full_spec((H, 2 * H)), _full_spec((1, 2 * H))],
        [_row_spec(_BN_ROWS, 2 * H), _full_spec((8, 2 * H))],
        [jax.ShapeDtypeStruct((N, 2 * H), _F32),
         jax.ShapeDtypeStruct((8, 2 * H), _F32)],
    )(h, aggr, eps_l, w1, b1)


def _run_tk2(z1, sums1, g1, bb1, w2, b2):
    return _tc_call(
        _tk2_body, N // _BN_ROWS,
        [_row_spec(_BN_ROWS, 2 * H), _full_spec((8, 2 * H)),
         _full_spec((1, 2 * H)), _full_spec((1, 2 * H)),
         _full_spec((2 * H, H)), _full_spec((1, H))],
        [_row_spec(_BN_ROWS, H), _full_spec((8, H))],
        [jax.ShapeDtypeStruct((N, H), _F32),
         jax.ShapeDtypeStruct((8, H), _F32)],
    )(z1, sums1, g1, bb1, w2, b2)


def _run_tk3(z2, sums2, g, b):
    return _tc_call1(
        _tk3_body, N // _BN_ROWS,
        [_row_spec(_BN_ROWS, H), _full_spec((8, H)), _full_spec((1, H)),
         _full_spec((1, H))],
        [_row_spec(_BN_ROWS, H)],
        [jax.ShapeDtypeStruct((N, H), _F32)],
    )(z2, sums2, g, b)


def _run_tk4(cm_sum, cnt_c, hc, w, b):
    return _tc_call(
        _tk4_body, NC // _BC_ROWS,
        [_row_spec(_BC_ROWS, H), _row_spec(_BC_ROWS, 16), _row_spec(_BC_ROWS, H),
         _full_spec((H, H)), _full_spec((1, H))],
        [_row_spec(_BC_ROWS, H), _full_spec((8, H))],
        [jax.ShapeDtypeStruct((NC, H), _F32),
         jax.ShapeDtypeStruct((8, H), _F32)],
    )(cm_sum, cnt_c, hc, w, b)


def _run_tk5(hc_pre, sums_s, g, b):
    return _tc_call1(
        _tk5_body, NC // _BC_ROWS,
        [_row_spec(_BC_ROWS, H), _full_spec((8, H)), _full_spec((1, H)),
         _full_spec((1, H))],
        [_row_spec(_BC_ROWS, H)],
        [jax.ShapeDtypeStruct((NC, H), _F32)],
    )(hc_pre, sums_s, g, b)


def _run_tk6(am_sum, cnt_a, h_mid, w, b):
    return _tc_call1(
        _tk6_body, N // _BN_ROWS,
        [_row_spec(_BN_ROWS, H), _row_spec(_BN_ROWS, 16), _row_spec(_BN_ROWS, H),
         _full_spec((H, H)), _full_spec((1, H))],
        [_row_spec(_BN_ROWS, H)],
        [jax.ShapeDtypeStruct((N, H), _F32)],
    )(am_sum, cnt_a, h_mid, w, b)


def kernel(x, x_clique, graph_lpe, edge_index_graph, edge_attr_graph,
           atom2clique_row, atom2clique_col,
           atom_emb, clique_emb, clique_W, clique_b, bond_emb, eps,
           W1, b1, bn1_g, bn1_b, W2, b2, gn_g, gn_b, sn_g, sn_b,
           a2c_W, a2c_b, c2a_W, c2a_b):
    i32 = jnp.int32
    f32 = jnp.float32

    x = x.astype(i32)
    t9 = atom_emb.reshape(9 * 100, H).astype(f32)
    idx9 = (x + 100 * jnp.arange(9, dtype=i32)[None, :]).T
    idx9 = jnp.pad(idx9, ((0, 0), (0, N_PAD - N))).reshape(9 * N_PAD)

    t4 = (clique_emb @ clique_W + clique_b).astype(f32)
    xc = jnp.pad(x_clique.astype(i32), (0, NC_PAD - NC))

    src = edge_index_graph[0].astype(i32)
    dst = edge_index_graph[1].astype(i32)
    ea = edge_attr_graph.astype(i32)
    ce = ea[:, 0] * 36 + ea[:, 1] * 6 + ea[:, 2]
    src_p = jnp.pad(src, (0, E_PAD - E))
    ce_p = jnp.pad(ce, (0, E_PAD - E))
    dst_p = jnp.pad(dst, (0, E_PAD - E), constant_values=-1)

    row = atom2clique_row.astype(i32)
    col = atom2clique_col.astype(i32)
    row_g = jnp.pad(row, (0, A_PAD - A))
    col_g = jnp.pad(col, (0, A_PAD - A))
    row_s = jnp.pad(row, (0, A_PAD - A), constant_values=-1)
    col_s = jnp.pad(col, (0, A_PAD - A), constant_values=-1)

    bts = [
        (bond_emb[l, 0][:, None, None, :] + bond_emb[l, 1][None, :, None, :]
         + bond_emb[l, 2][None, None, :, :]).reshape(216, H).astype(f32)
        for l in range(3)
    ]

    zeros_n = jnp.zeros((1568, H), f32)
    zeros_cnt = jnp.zeros((3128, 16), f32)
    ones16 = jnp.ones((CH, 16), f32)

    h0_pad, hc0_pad = _encode_sc(t9, idx9, t4, xc)
    h = h0_pad[:N]
    hc = hc0_pad[:NC]
    cnt_a, cnt_c = _counts_sc(row_s, col_s, ones16, zeros_cnt)

    for l in range(3):
        eps_l = jnp.full((8, 128), eps[l], f32)
        aggr = _edge_sc(h, bts[l], src_p, ce_p, dst_p, zeros_n)
        z1, sums1 = _run_tk1(h, aggr, eps_l, W1[l],
                             b1[l].reshape(1, 2 * H))
        z2, sums2 = _run_tk2(z1, sums1, bn1_g[l].reshape(1, 2 * H),
                             bn1_b[l].reshape(1, 2 * H), W2[l],
                             b2[l].reshape(1, H))
        h_mid = _run_tk3(z2, sums2, gn_g[l].reshape(1, H),
                         gn_b[l].reshape(1, H))
        cm_sum = _a2c_sc(h_mid, row_g, col_s, zeros_n)
        hc_pre, sums_s = _run_tk4(cm_sum, cnt_c, hc, a2c_W[l],
                                  a2c_b[l].reshape(1, H))
        hc = _run_tk5(hc_pre, sums_s, sn_g[l].reshape(1, H),
                      sn_b[l].reshape(1, H))
        am_sum = _c2a_sc(hc, col_g, row_s, zeros_n)
        h = _run_tk6(am_sum, cnt_a, h_mid, c2a_W[l],
                     c2a_b[l].reshape(1, H))

    return h


# speedup vs baseline: 4.4279x; 1.4855x over previous
import functools

import jax
import jax.numpy as jnp
from jax import lax
from jax.experimental import pallas as pl
from jax.experimental.pallas import tpu as pltpu
from jax.experimental.pallas import tpu_sc as plsc

N = 50000
NC = 25000
E = 800000
A = 50000
H = 64

NCORE = 2
NSUB = 16
CH = 128

E_PAD = 800768
CE = 64
A_PAD = 51200
N_PAD = 50048
NC_PAD = 25088

HALF_N = N // 2
HALF_NC = NC // 2
SP_N = 25088
SP_NC = 12544

_MESH = plsc.VectorSubcoreMesh(core_axis_name="c", subcore_axis_name="s")
_SC_PARAMS = pltpu.CompilerParams(use_tc_tiling_on_sc=False)


def _relu_add_inplace(abuf, bbuf, rows):
    @pl.loop(0, rows, unroll=8)
    def _(r):
        for q in range(H // 16):
            sl = pl.ds(q * 16, 16)
            abuf[r, sl] = jnp.maximum(abuf[r, sl] + bbuf[r, sl], 0.0)


def _local_idx(dv, lo, half):
    valid = (dv >= lo) & (dv < lo + half)
    return jnp.where(valid, dv - lo, -1)


@functools.partial(
    pl.kernel,
    out_type=(
        jax.ShapeDtypeStruct((N_PAD, H), jnp.float32),
        jax.ShapeDtypeStruct((NC_PAD, H), jnp.float32),
    ),
    mesh=_MESH,
    compiler_params=_SC_PARAMS,
    scratch_types=[
        pltpu.VMEM((CH,), jnp.int32),
        pltpu.VMEM((CH, H), jnp.float32),
        pltpu.VMEM((CH, H), jnp.float32),
    ],
)
def _encode_sc(t9_hbm, idx9_hbm, t4_hbm, xc_hbm, h0_out, hc0_out,
               idxv, acc, tmp):
    w = lax.axis_index("s") * NCORE + lax.axis_index("c")

    n_chunks = N_PAD // CH

    @pl.loop(0, 13)
    def _(jj):
        j = w * 13 + jj

        @pl.when(j < n_chunks)
        def _():
            b = j * CH
            pltpu.sync_copy(idx9_hbm.at[pl.ds(b, CH)], idxv)
            pltpu.sync_copy(t9_hbm.at[idxv], acc)
            for i in range(1, 9):
                pltpu.sync_copy(idx9_hbm.at[pl.ds(i * N_PAD + b, CH)], idxv)
                pltpu.sync_copy(t9_hbm.at[idxv], tmp)

                @pl.loop(0, CH, unroll=8)
                def _(r):
                    for q in range(H // 16):
                        sl = pl.ds(q * 16, 16)
                        acc[r, sl] = acc[r, sl] + tmp[r, sl]
            pltpu.sync_copy(acc, h0_out.at[pl.ds(b, CH), :])

    c_chunks = NC_PAD // CH

    @pl.loop(0, 7)
    def _(jj):
        j = w * 7 + jj

        @pl.when(j < c_chunks)
        def _():
            b = j * CH
            pltpu.sync_copy(xc_hbm.at[pl.ds(b, CH)], idxv)
            pltpu.sync_copy(t4_hbm.at[idxv], acc)
            pltpu.sync_copy(acc, hc0_out.at[pl.ds(b, CH), :])


@functools.partial(
    pl.kernel,
    out_type=(
        jax.ShapeDtypeStruct((N, 16), jnp.float32),
        jax.ShapeDtypeStruct((NC, 16), jnp.float32),
    ),
    mesh=_MESH,
    compiler_params=_SC_PARAMS,
    scratch_types=[
        pltpu.VMEM((CH,), jnp.int32),
        pltpu.VMEM((CH, 16), jnp.float32),
        pltpu.VMEM_SHARED((N_PAD, 16), jnp.float32),
    ],
)
def _counts_sc(row_s_hbm, col_s_hbm, ones_hbm, zc_hbm, cnt_a_out, cnt_c_out,
               idxv, ones_v, cnt_sh):
    c = lax.axis_index("c")
    s = lax.axis_index("s")
    pltpu.sync_copy(ones_hbm, ones_v)

    @pl.when(c == 0)
    def _():
        pltpu.sync_copy(zc_hbm, cnt_sh.at[pl.ds(s * 3128, 3128), :])

    @pl.when(c == 1)
    def _():
        pltpu.sync_copy(zc_hbm.at[pl.ds(0, 1564), :],
                        cnt_sh.at[pl.ds(s * 1564, 1564), :])

    plsc.subcore_barrier()

    n_chunks = A_PAD // (NSUB * CH)

    @pl.loop(0, n_chunks)
    def _(j):
        b = s * (A_PAD // NSUB) + j * CH

        @pl.when(c == 0)
        def _():
            pltpu.sync_copy(row_s_hbm.at[pl.ds(b, CH)], idxv)
            pltpu.sync_copy(
                ones_v, cnt_sh.at[plsc.Indices(idxv, ignored_value=-1)],
                add=True)

        @pl.when(c == 1)
        def _():
            pltpu.sync_copy(col_s_hbm.at[pl.ds(b, CH)], idxv)
            pltpu.sync_copy(
                ones_v, cnt_sh.at[plsc.Indices(idxv, ignored_value=-1)],
                add=True)

    plsc.subcore_barrier()

    @pl.when(c == 0)
    def _():
        @pl.when(s < 15)
        def _():
            pltpu.sync_copy(cnt_sh.at[pl.ds(s * 3128, 3128), :],
                            cnt_a_out.at[pl.ds(s * 3128, 3128), :])

        @pl.when(s == 15)
        def _():
            pltpu.sync_copy(cnt_sh.at[pl.ds(15 * 3128, 3080), :],
                            cnt_a_out.at[pl.ds(15 * 3128, 3080), :])

    @pl.when(c == 1)
    def _():
        @pl.when(s < 15)
        def _():
            pltpu.sync_copy(cnt_sh.at[pl.ds(s * 1564, 1564), :],
                            cnt_c_out.at[pl.ds(s * 1564, 1564), :])

        @pl.when(s == 15)
        def _():
            pltpu.sync_copy(cnt_sh.at[pl.ds(15 * 1564, 1540), :],
                            cnt_c_out.at[pl.ds(15 * 1564, 1540), :])


@functools.partial(
    pl.kernel,
    out_type=jax.ShapeDtypeStruct((N, H), jnp.float32),
    mesh=_MESH,
    compiler_params=_SC_PARAMS,
    scratch_types=[
        [pltpu.VMEM((CE,), jnp.int32)] * 2,
        [pltpu.VMEM((CE,), jnp.int32)] * 2,
        [pltpu.VMEM((CE,), jnp.int32)] * 2,
        [pltpu.VMEM((CE,), jnp.int32)] * 2,
        [pltpu.VMEM((CE, H), jnp.float32)] * 2,
        [pltpu.VMEM((CE, H), jnp.float32)] * 2,
        [pltpu.SemaphoreType.DMA] * 2,
        [pltpu.SemaphoreType.DMA] * 2,
        [pltpu.SemaphoreType.DMA] * 2,
        pltpu.VMEM_SHARED((SP_N, H), jnp.float32),
    ],
)
def _edge_sc(h_hbm, bt_hbm, src_hbm, ce_hbm, dst_hbm, z_hbm, aggr_out,
             srcv, cev, dstv, dlv, hbuf, ebuf, sem_idx, sem_g, sem_s,
             aggr_sh):
    c = lax.axis_index("c")
    s = lax.axis_index("s")
    lo = c * HALF_N

    pltpu.sync_copy(z_hbm.at[pl.ds(0, 1568), :],
                    aggr_sh.at[pl.ds(s * 1568, 1568), :])
    plsc.subcore_barrier()

    per_sub = E_PAD // NSUB
    n_chunks = per_sub // CE

    def fire_idx(b, j):
        bb = s * per_sub + j * CE
        pltpu.async_copy(src_hbm.at[pl.ds(bb, CE)], srcv[b], sem_idx[b])
        pltpu.async_copy(ce_hbm.at[pl.ds(bb, CE)], cev[b], sem_idx[b])
        pltpu.async_copy(dst_hbm.at[pl.ds(bb, CE)], dstv[b], sem_idx[b])

    def wait_idx(b):
        pltpu.make_async_copy(src_hbm.at[pl.ds(0, CE)], srcv[b],
                              sem_idx[b]).wait()
        pltpu.make_async_copy(ce_hbm.at[pl.ds(0, CE)], cev[b],
                              sem_idx[b]).wait()
        pltpu.make_async_copy(dst_hbm.at[pl.ds(0, CE)], dstv[b],
                              sem_idx[b]).wait()

    def fire_gathers(b):
        pltpu.async_copy(h_hbm.at[srcv[b]], hbuf[b], sem_g[b])
        pltpu.async_copy(bt_hbm.at[cev[b]], ebuf[b], sem_g[b])

    def wait_gathers(b):
        pltpu.make_async_copy(h_hbm.at[srcv[b]], hbuf[b], sem_g[b]).wait()
        pltpu.make_async_copy(bt_hbm.at[cev[b]], ebuf[b], sem_g[b]).wait()

    def fire_scatter(b):
        _relu_add_inplace(hbuf[b], ebuf[b], CE)
        for q in range(CE // 16):
            sl = pl.ds(q * 16, 16)
            dlv[b][sl] = _local_idx(dstv[b][sl], lo, HALF_N)
        pltpu.async_copy(
            hbuf[b], aggr_sh.at[plsc.Indices(dlv[b], ignored_value=-1)],
            sem_s[b], add=True)

    def wait_scatter(b):
        pltpu.make_async_copy(
            hbuf[b], aggr_sh.at[plsc.Indices(dlv[b], ignored_value=-1)],
            sem_s[b]).wait()

    fire_idx(0, 0)
    wait_idx(0)
    fire_gathers(0)
    fire_idx(1, 1)

    @pl.loop(0, n_chunks)
    def _(j):
        def body(b, nb):
            @pl.when(j + 1 < n_chunks)
            def _():
                wait_idx(nb)

                @pl.when(j >= 1)
                def _():
                    wait_scatter(nb)

                fire_gathers(nb)

            wait_gathers(b)
            fire_scatter(b)

            @pl.when(j + 2 < n_chunks)
            def _():
                fire_idx(b, j + 2)

        @pl.when(j % 2 == 0)
        def _():
            body(0, 1)

        @pl.when(j % 2 == 1)
        def _():
            body(1, 0)

    wait_scatter(0)
    wait_scatter(1)
    plsc.subcore_barrier()

    @pl.when(s < 15)
    def _():
        pltpu.sync_copy(aggr_sh.at[pl.ds(s * 1568, 1568), :],
                        aggr_out.at[pl.ds(lo + s * 1568, 1568), :])

    @pl.when(s == 15)
    def _():
        pltpu.sync_copy(aggr_sh.at[pl.ds(15 * 1568, 1480), :],
                        aggr_out.at[pl.ds(lo + 15 * 1568, 1480), :])


def _make_gss(table_rows, out_rows, sp_rows):
    half = out_rows // 2
    span = sp_rows // NSUB
    last = half - 15 * span
    n_chunks = A_PAD // (NSUB * CH)

    @functools.partial(
        pl.kernel,
        out_type=jax.ShapeDtypeStruct((out_rows, H), jnp.float32),
        mesh=_MESH,
        compiler_params=_SC_PARAMS,
        scratch_types=[
            pltpu.VMEM((CH,), jnp.int32),
            pltpu.VMEM((CH,), jnp.int32),
            pltpu.VMEM((CH,), jnp.int32),
            pltpu.VMEM((CH, H), jnp.float32),
            pltpu.VMEM_SHARED((sp_rows, H), jnp.float32),
        ],
    )
    def gss(table_hbm, gidx_hbm, sidx_hbm, z_hbm, out_hbm,
            gv, sv, dlv, buf, acc_sh):
        c = lax.axis_index("c")
        s = lax.axis_index("s")
        lo = c * half

        pltpu.sync_copy(z_hbm.at[pl.ds(0, span), :],
                        acc_sh.at[pl.ds(s * span, span), :])
        plsc.subcore_barrier()

        @pl.loop(0, n_chunks)
        def _(j):
            b = s * (A_PAD // NSUB) + j * CH
            pltpu.sync_copy(gidx_hbm.at[pl.ds(b, CH)], gv)
            pltpu.sync_copy(sidx_hbm.at[pl.ds(b, CH)], sv)
            pltpu.sync_copy(table_hbm.at[gv], buf)
            for k in range(CH // 16):
                sl = pl.ds(k * 16, 16)
                dlv[sl] = _local_idx(sv[sl], lo, half)
            pltpu.sync_copy(
                buf, acc_sh.at[plsc.Indices(dlv, ignored_value=-1)], add=True)

        plsc.subcore_barrier()

        @pl.when(s < 15)
        def _():
            pltpu.sync_copy(acc_sh.at[pl.ds(s * span, span), :],
                            out_hbm.at[pl.ds(lo + s * span, span), :])

        @pl.when(s == 15)
        def _():
            pltpu.sync_copy(acc_sh.at[pl.ds(15 * span, last), :],
                            out_hbm.at[pl.ds(lo + 15 * span, last), :])

    return gss


_a2c_sc = _make_gss(N, NC, SP_NC)
_c2a_sc = _make_gss(NC, N, SP_N)


_BN_EPS = 1e-5
_F32 = jnp.float32


def _dot(a, b):
    return jnp.dot(a, b, preferred_element_type=_F32)


def _stats_update(sums_ref, z, i):
    @pl.when(i == 0)
    def _():
        sums_ref[...] = jnp.zeros_like(sums_ref)

    sums_ref[0:1, :] += jnp.sum(z, axis=0, keepdims=True)
    sums_ref[1:2, :] += jnp.sum(z * z, axis=0, keepdims=True)


def _bn_apply(z, sums, nrows, g, b):
    m = sums[0:1, :] / nrows
    var = sums[1:2, :] / nrows - m * m
    return (z - m) * lax.rsqrt(var + _BN_EPS) * g + b


def _tk1_body(h_ref, aggr_ref, eps_ref, w1_ref, b1_ref, z1_ref, sums_ref):
    i = pl.program_id(0)
    u = (1.0 + eps_ref[0, 0]) * h_ref[...] + aggr_ref[...]
    z = _dot(u, w1_ref[...]) + b1_ref[...]
    z1_ref[...] = z
    _stats_update(sums_ref, z, i)


def _tk2_body(z1_ref, sums1_ref, g1_ref, bb1_ref, w2_ref, b2_ref,
              z2_ref, sums_ref):
    i = pl.program_id(0)
    v = jax.nn.relu(_bn_apply(z1_ref[...], sums1_ref[...], float(N),
                              g1_ref[...], bb1_ref[...]))
    z = _dot(v, w2_ref[...]) + b2_ref[...]
    z2_ref[...] = z
    _stats_update(sums_ref, z, i)


def _tk3_body(z2_ref, sums2_ref, g_ref, b_ref, h_ref):
    h_ref[...] = jax.nn.relu(_bn_apply(z2_ref[...], sums2_ref[...], float(N),
                                       g_ref[...], b_ref[...]))


def _tk4_body(cm_ref, cnt_ref, hc_ref, w_ref, b_ref, out_ref, sums_ref):
    i = pl.program_id(0)
    cm = cm_ref[...] / jnp.maximum(cnt_ref[:, 0:1], 1.0)
    z = hc_ref[...] + jax.nn.relu(_dot(cm, w_ref[...]) + b_ref[...])
    out_ref[...] = z
    _stats_update(sums_ref, z, i)


def _tk5_body(zp_ref, sums_ref, g_ref, b_ref, out_ref):
    out_ref[...] = jax.nn.relu(_bn_apply(zp_ref[...], sums_ref[...], float(NC),
                                         g_ref[...], b_ref[...]))


def _tk6_body(am_ref, cnt_ref, h_ref, w_ref, b_ref, out_ref):
    am = am_ref[...] / jnp.maximum(cnt_ref[:, 0:1], 1.0)
    out_ref[...] = h_ref[...] + jax.nn.relu(_dot(am, w_ref[...]) + b_ref[...])


def _row_spec(bs, cols):
    return pl.BlockSpec((bs, cols), lambda i: (i, 0))


def _full_spec(shape):
    return pl.BlockSpec(shape, lambda i: tuple(0 for _ in shape))


_BN_ROWS = 2000
_BC_ROWS = 1000


def _tc_call(body, grid, in_specs, out_specs, out_shapes):
    return pl.pallas_call(
        body, grid=(grid,), in_specs=in_specs, out_specs=out_specs,
        out_shape=out_shapes)


def _tc_call1(*args):
    def run(*ins):
        (out,) = _tc_call(*args)(*ins)
        return out
    return run


def _run_tk1(h, aggr, eps_l, w1, b1):
    return _tc_call(
        _tk1_body, N // _BN_ROWS,
        [_row_spec(_BN_ROWS, H), _row_spec(_BN_ROWS, H), _full_spec((8, 128)),
         _full_spec((H, 2 * H)), _full_spec((1, 2 * H))],
        [_row_spec(_BN_ROWS, 2 * H), _full_spec((8, 2 * H))],
        [jax.ShapeDtypeStruct((N, 2 * H), _F32),
         jax.ShapeDtypeStruct((8, 2 * H), _F32)],
    )(h, aggr, eps_l, w1, b1)


def _run_tk2(z1, sums1, g1, bb1, w2, b2):
    return _tc_call(
        _tk2_body, N // _BN_ROWS,
        [_row_spec(_BN_ROWS, 2 * H), _full_spec((8, 2 * H)),
         _full_spec((1, 2 * H)), _full_spec((1, 2 * H)),
         _full_spec((2 * H, H)), _full_spec((1, H))],
        [_row_spec(_BN_ROWS, H), _full_spec((8, H))],
        [jax.ShapeDtypeStruct((N, H), _F32),
         jax.ShapeDtypeStruct((8, H), _F32)],
    )(z1, sums1, g1, bb1, w2, b2)


def _run_tk3(z2, sums2, g, b):
    return _tc_call1(
        _tk3_body, N // _BN_ROWS,
        [_row_spec(_BN_ROWS, H), _full_spec((8, H)), _full_spec((1, H)),
         _full_spec((1, H))],
        [_row_spec(_BN_ROWS, H)],
        [jax.ShapeDtypeStruct((N, H), _F32)],
    )(z2, sums2, g, b)


def _run_tk4(cm_sum, cnt_c, hc, w, b):
    return _tc_call(
        _tk4_body, NC // _BC_ROWS,
        [_row_spec(_BC_ROWS, H), _row_spec(_BC_ROWS, 16), _row_spec(_BC_ROWS, H),
         _full_spec((H, H)), _full_spec((1, H))],
        [_row_spec(_BC_ROWS, H), _full_spec((8, H))],
        [jax.ShapeDtypeStruct((NC, H), _F32),
         jax.ShapeDtypeStruct((8, H), _F32)],
    )(cm_sum, cnt_c, hc, w, b)


def _run_tk5(hc_pre, sums_s, g, b):
    return _tc_call1(
        _tk5_body, NC // _BC_ROWS,
        [_row_spec(_BC_ROWS, H), _full_spec((8, H)), _full_spec((1, H)),
         _full_spec((1, H))],
        [_row_spec(_BC_ROWS, H)],
        [jax.ShapeDtypeStruct((NC, H), _F32)],
    )(hc_pre, sums_s, g, b)


def _run_tk6(am_sum, cnt_a, h_mid, w, b):
    return _tc_call1(
        _tk6_body, N // _BN_ROWS,
        [_row_spec(_BN_ROWS, H), _row_spec(_BN_ROWS, 16), _row_spec(_BN_ROWS, H),
         _full_spec((H, H)), _full_spec((1, H))],
        [_row_spec(_BN_ROWS, H)],
        [jax.ShapeDtypeStruct((N, H), _F32)],
    )(am_sum, cnt_a, h_mid, w, b)


def kernel(x, x_clique, graph_lpe, edge_index_graph, edge_attr_graph,
           atom2clique_row, atom2clique_col,
           atom_emb, clique_emb, clique_W, clique_b, bond_emb, eps,
           W1, b1, bn1_g, bn1_b, W2, b2, gn_g, gn_b, sn_g, sn_b,
           a2c_W, a2c_b, c2a_W, c2a_b):
    i32 = jnp.int32
    f32 = jnp.float32

    x = x.astype(i32)
    t9 = atom_emb.reshape(9 * 100, H).astype(f32)
    idx9 = (x + 100 * jnp.arange(9, dtype=i32)[None, :]).T
    idx9 = jnp.pad(idx9, ((0, 0), (0, N_PAD - N))).reshape(9 * N_PAD)

    t4 = (clique_emb @ clique_W + clique_b).astype(f32)
    xc = jnp.pad(x_clique.astype(i32), (0, NC_PAD - NC))

    src = edge_index_graph[0].astype(i32)
    dst = edge_index_graph[1].astype(i32)
    ea = edge_attr_graph.astype(i32)
    ce = ea[:, 0] * 36 + ea[:, 1] * 6 + ea[:, 2]
    src_p = jnp.pad(src, (0, E_PAD - E))
    ce_p = jnp.pad(ce, (0, E_PAD - E))
    dst_p = jnp.pad(dst, (0, E_PAD - E), constant_values=-1)

    row = atom2clique_row.astype(i32)
    col = atom2clique_col.astype(i32)
    row_g = jnp.pad(row, (0, A_PAD - A))
    col_g = jnp.pad(col, (0, A_PAD - A))
    row_s = jnp.pad(row, (0, A_PAD - A), constant_values=-1)
    col_s = jnp.pad(col, (0, A_PAD - A), constant_values=-1)

    bts = [
        (bond_emb[l, 0][:, None, None, :] + bond_emb[l, 1][None, :, None, :]
         + bond_emb[l, 2][None, None, :, :]).reshape(216, H).astype(f32)
        for l in range(3)
    ]

    zeros_n = jnp.zeros((1568, H), f32)
    zeros_cnt = jnp.zeros((3128, 16), f32)
    ones16 = jnp.ones((CH, 16), f32)

    h0_pad, hc0_pad = _encode_sc(t9, idx9, t4, xc)
    h = h0_pad[:N]
    hc = hc0_pad[:NC]
    cnt_a, cnt_c = _counts_sc(row_s, col_s, ones16, zeros_cnt)

    for l in range(3):
        eps_l = jnp.full((8, 128), eps[l], f32)
        aggr = _edge_sc(h, bts[l], src_p, ce_p, dst_p, zeros_n)
        z1, sums1 = _run_tk1(h, aggr, eps_l, W1[l],
                             b1[l].reshape(1, 2 * H))
        z2, sums2 = _run_tk2(z1, sums1, bn1_g[l].reshape(1, 2 * H),
                             bn1_b[l].reshape(1, 2 * H), W2[l],
                             b2[l].reshape(1, H))
        h_mid = _run_tk3(z2, sums2, gn_g[l].reshape(1, H),
                         gn_b[l].reshape(1, H))
        cm_sum = _a2c_sc(h_mid, row_g, col_s, zeros_n)
        hc_pre, sums_s = _run_tk4(cm_sum, cnt_c, hc, a2c_W[l],
                                  a2c_b[l].reshape(1, H))
        hc = _run_tk5(hc_pre, sums_s, sn_g[l].reshape(1, H),
                      sn_b[l].reshape(1, H))
        am_sum = _c2a_sc(hc, col_g, row_s, zeros_n)
        h = _run_tk6(am_sum, cnt_a, h_mid, c2a_W[l],
                     c2a_b[l].reshape(1, H))

    return h


# speedup vs baseline: 4.4388x; 1.0024x over previous
import functools

import jax
import jax.numpy as jnp
from jax import lax
from jax.experimental import pallas as pl
from jax.experimental.pallas import tpu as pltpu
from jax.experimental.pallas import tpu_sc as plsc

N = 50000
NC = 25000
E = 800000
A = 50000
H = 64

NCORE = 2
NSUB = 16
CH = 128

E_PAD = 800768
CE = 64
A_PAD = 51200
N_PAD = 50048
NC_PAD = 25088

HALF_N = N // 2
HALF_NC = NC // 2
SP_N = 25088
SP_NC = 12544

_MESH = plsc.VectorSubcoreMesh(core_axis_name="c", subcore_axis_name="s")
_SC_PARAMS = pltpu.CompilerParams(use_tc_tiling_on_sc=False)


def _relu_add_inplace(abuf, bbuf, rows):
    @pl.loop(0, rows, unroll=8)
    def _(r):
        for q in range(H // 16):
            sl = pl.ds(q * 16, 16)
            abuf[r, sl] = jnp.maximum(abuf[r, sl] + bbuf[r, sl], 0.0)


def _local_idx(dv, lo, half):
    valid = (dv >= lo) & (dv < lo + half)
    return jnp.where(valid, dv - lo, -1)


@functools.partial(
    pl.kernel,
    out_type=(
        jax.ShapeDtypeStruct((N_PAD, H), jnp.float32),
        jax.ShapeDtypeStruct((NC_PAD, H), jnp.float32),
    ),
    mesh=_MESH,
    compiler_params=_SC_PARAMS,
    scratch_types=[
        pltpu.VMEM((CH,), jnp.int32),
        pltpu.VMEM((CH, H), jnp.float32),
        pltpu.VMEM((CH, H), jnp.float32),
    ],
)
def _encode_sc(t9_hbm, idx9_hbm, t4_hbm, xc_hbm, h0_out, hc0_out,
               idxv, acc, tmp):
    w = lax.axis_index("s") * NCORE + lax.axis_index("c")

    n_chunks = N_PAD // CH

    @pl.loop(0, 13)
    def _(jj):
        j = w * 13 + jj

        @pl.when(j < n_chunks)
        def _():
            b = j * CH
            pltpu.sync_copy(idx9_hbm.at[pl.ds(b, CH)], idxv)
            pltpu.sync_copy(t9_hbm.at[idxv], acc)
            for i in range(1, 9):
                pltpu.sync_copy(idx9_hbm.at[pl.ds(i * N_PAD + b, CH)], idxv)
                pltpu.sync_copy(t9_hbm.at[idxv], tmp)

                @pl.loop(0, CH, unroll=8)
                def _(r):
                    for q in range(H // 16):
                        sl = pl.ds(q * 16, 16)
                        acc[r, sl] = acc[r, sl] + tmp[r, sl]
            pltpu.sync_copy(acc, h0_out.at[pl.ds(b, CH), :])

    c_chunks = NC_PAD // CH

    @pl.loop(0, 7)
    def _(jj):
        j = w * 7 + jj

        @pl.when(j < c_chunks)
        def _():
            b = j * CH
            pltpu.sync_copy(xc_hbm.at[pl.ds(b, CH)], idxv)
            pltpu.sync_copy(t4_hbm.at[idxv], acc)
            pltpu.sync_copy(acc, hc0_out.at[pl.ds(b, CH), :])


@functools.partial(
    pl.kernel,
    out_type=(
        jax.ShapeDtypeStruct((N, 16), jnp.float32),
        jax.ShapeDtypeStruct((NC, 16), jnp.float32),
    ),
    mesh=_MESH,
    compiler_params=_SC_PARAMS,
    scratch_types=[
        pltpu.VMEM((CH,), jnp.int32),
        pltpu.VMEM((CH, 16), jnp.float32),
        pltpu.VMEM_SHARED((N_PAD, 16), jnp.float32),
    ],
)
def _counts_sc(row_s_hbm, col_s_hbm, ones_hbm, zc_hbm, cnt_a_out, cnt_c_out,
               idxv, ones_v, cnt_sh):
    c = lax.axis_index("c")
    s = lax.axis_index("s")
    pltpu.sync_copy(ones_hbm, ones_v)

    @pl.when(c == 0)
    def _():
        pltpu.sync_copy(zc_hbm, cnt_sh.at[pl.ds(s * 3128, 3128), :])

    @pl.when(c == 1)
    def _():
        pltpu.sync_copy(zc_hbm.at[pl.ds(0, 1564), :],
                        cnt_sh.at[pl.ds(s * 1564, 1564), :])

    plsc.subcore_barrier()

    n_chunks = A_PAD // (NSUB * CH)

    @pl.loop(0, n_chunks)
    def _(j):
        b = s * (A_PAD // NSUB) + j * CH

        @pl.when(c == 0)
        def _():
            pltpu.sync_copy(row_s_hbm.at[pl.ds(b, CH)], idxv)
            pltpu.sync_copy(
                ones_v, cnt_sh.at[plsc.Indices(idxv, ignored_value=-1)],
                add=True)

        @pl.when(c == 1)
        def _():
            pltpu.sync_copy(col_s_hbm.at[pl.ds(b, CH)], idxv)
            pltpu.sync_copy(
                ones_v, cnt_sh.at[plsc.Indices(idxv, ignored_value=-1)],
                add=True)

    plsc.subcore_barrier()

    @pl.when(c == 0)
    def _():
        @pl.when(s < 15)
        def _():
            pltpu.sync_copy(cnt_sh.at[pl.ds(s * 3128, 3128), :],
                            cnt_a_out.at[pl.ds(s * 3128, 3128), :])

        @pl.when(s == 15)
        def _():
            pltpu.sync_copy(cnt_sh.at[pl.ds(15 * 3128, 3080), :],
                            cnt_a_out.at[pl.ds(15 * 3128, 3080), :])

    @pl.when(c == 1)
    def _():
        @pl.when(s < 15)
        def _():
            pltpu.sync_copy(cnt_sh.at[pl.ds(s * 1564, 1564), :],
                            cnt_c_out.at[pl.ds(s * 1564, 1564), :])

        @pl.when(s == 15)
        def _():
            pltpu.sync_copy(cnt_sh.at[pl.ds(15 * 1564, 1540), :],
                            cnt_c_out.at[pl.ds(15 * 1564, 1540), :])


@functools.partial(
    pl.kernel,
    out_type=jax.ShapeDtypeStruct((N, H), jnp.float32),
    mesh=_MESH,
    compiler_params=_SC_PARAMS,
    scratch_types=[
        [pltpu.VMEM((CE,), jnp.int32)] * 2,
        [pltpu.VMEM((CE,), jnp.int32)] * 2,
        [pltpu.VMEM((CE,), jnp.int32)] * 2,
        [pltpu.VMEM((CE,), jnp.int32)] * 2,
        [pltpu.VMEM((CE, H), jnp.float32)] * 2,
        [pltpu.VMEM((CE, H), jnp.float32)] * 2,
        [pltpu.SemaphoreType.DMA] * 2,
        [pltpu.SemaphoreType.DMA] * 2,
        [pltpu.SemaphoreType.DMA] * 2,
        pltpu.VMEM_SHARED((SP_N, H), jnp.float32),
    ],
)
def _edge_sc(h_hbm, bt_hbm, src_hbm, ce_hbm, dst_hbm, z_hbm, aggr_out,
             srcv, cev, dstv, dlv, hbuf, ebuf, sem_idx, sem_g, sem_s,
             aggr_sh):
    c = lax.axis_index("c")
    s = lax.axis_index("s")
    lo = c * HALF_N

    pltpu.sync_copy(z_hbm.at[pl.ds(0, 1568), :],
                    aggr_sh.at[pl.ds(s * 1568, 1568), :])
    plsc.subcore_barrier()

    per_sub = E_PAD // NSUB
    n_chunks = per_sub // CE

    def fire_idx(b, j):
        bb = s * per_sub + j * CE
        pltpu.async_copy(src_hbm.at[pl.ds(bb, CE)], srcv[b], sem_idx[b])
        pltpu.async_copy(ce_hbm.at[pl.ds(bb, CE)], cev[b], sem_idx[b])
        pltpu.async_copy(dst_hbm.at[pl.ds(bb, CE)], dstv[b], sem_idx[b])

    def wait_idx(b):
        pltpu.make_async_copy(src_hbm.at[pl.ds(0, CE)], srcv[b],
                              sem_idx[b]).wait()
        pltpu.make_async_copy(ce_hbm.at[pl.ds(0, CE)], cev[b],
                              sem_idx[b]).wait()
        pltpu.make_async_copy(dst_hbm.at[pl.ds(0, CE)], dstv[b],
                              sem_idx[b]).wait()
        for q in range(CE // 16):
            sl = pl.ds(q * 16, 16)
            dl = _local_idx(dstv[b][sl], lo, HALF_N)
            dlv[b][sl] = dl
            srcv[b][sl] = jnp.where(dl >= 0, srcv[b][sl], -1)
            cev[b][sl] = jnp.where(dl >= 0, cev[b][sl], -1)

    def fire_gathers(b):
        pltpu.async_copy(h_hbm.at[plsc.Indices(srcv[b], ignored_value=-1)],
                         hbuf[b], sem_g[b])
        pltpu.async_copy(bt_hbm.at[plsc.Indices(cev[b], ignored_value=-1)],
                         ebuf[b], sem_g[b])

    def wait_gathers(b):
        pltpu.make_async_copy(
            h_hbm.at[plsc.Indices(srcv[b], ignored_value=-1)], hbuf[b],
            sem_g[b]).wait()
        pltpu.make_async_copy(
            bt_hbm.at[plsc.Indices(cev[b], ignored_value=-1)], ebuf[b],
            sem_g[b]).wait()

    def fire_scatter(b):
        _relu_add_inplace(hbuf[b], ebuf[b], CE)
        pltpu.async_copy(
            hbuf[b], aggr_sh.at[plsc.Indices(dlv[b], ignored_value=-1)],
            sem_s[b], add=True)

    def wait_scatter(b):
        pltpu.make_async_copy(
            hbuf[b], aggr_sh.at[plsc.Indices(dlv[b], ignored_value=-1)],
            sem_s[b]).wait()

    fire_idx(0, 0)
    wait_idx(0)
    fire_gathers(0)
    fire_idx(1, 1)

    @pl.loop(0, n_chunks)
    def _(j):
        def body(b, nb):
            @pl.when(j + 1 < n_chunks)
            def _():
                @pl.when(j >= 1)
                def _():
                    wait_scatter(nb)

                wait_idx(nb)
                fire_gathers(nb)

            wait_gathers(b)
            fire_scatter(b)

            @pl.when(j + 2 < n_chunks)
            def _():
                fire_idx(b, j + 2)

        @pl.when(j % 2 == 0)
        def _():
            body(0, 1)

        @pl.when(j % 2 == 1)
        def _():
            body(1, 0)

    wait_scatter(0)
    wait_scatter(1)
    plsc.subcore_barrier()

    @pl.when(s < 15)
    def _():
        pltpu.sync_copy(aggr_sh.at[pl.ds(s * 1568, 1568), :],
                        aggr_out.at[pl.ds(lo + s * 1568, 1568), :])

    @pl.when(s == 15)
    def _():
        pltpu.sync_copy(aggr_sh.at[pl.ds(15 * 1568, 1480), :],
                        aggr_out.at[pl.ds(lo + 15 * 1568, 1480), :])


def _make_gss(table_rows, out_rows, sp_rows):
    half = out_rows // 2
    span = sp_rows // NSUB
    last = half - 15 * span
    n_chunks = A_PAD // (NSUB * CH)

    @functools.partial(
        pl.kernel,
        out_type=jax.ShapeDtypeStruct((out_rows, H), jnp.float32),
        mesh=_MESH,
        compiler_params=_SC_PARAMS,
        scratch_types=[
            pltpu.VMEM((CH,), jnp.int32),
            pltpu.VMEM((CH,), jnp.int32),
            pltpu.VMEM((CH,), jnp.int32),
            pltpu.VMEM((CH, H), jnp.float32),
            pltpu.VMEM_SHARED((sp_rows, H), jnp.float32),
        ],
    )
    def gss(table_hbm, gidx_hbm, sidx_hbm, z_hbm, out_hbm,
            gv, sv, dlv, buf, acc_sh):
        c = lax.axis_index("c")
        s = lax.axis_index("s")
        lo = c * half

        pltpu.sync_copy(z_hbm.at[pl.ds(0, span), :],
                        acc_sh.at[pl.ds(s * span, span), :])
        plsc.subcore_barrier()

        @pl.loop(0, n_chunks)
        def _(j):
            b = s * (A_PAD // NSUB) + j * CH
            pltpu.sync_copy(gidx_hbm.at[pl.ds(b, CH)], gv)
            pltpu.sync_copy(sidx_hbm.at[pl.ds(b, CH)], sv)
            pltpu.sync_copy(table_hbm.at[gv], buf)
            for k in range(CH // 16):
                sl = pl.ds(k * 16, 16)
                dlv[sl] = _local_idx(sv[sl], lo, half)
            pltpu.sync_copy(
                buf, acc_sh.at[plsc.Indices(dlv, ignored_value=-1)], add=True)

        plsc.subcore_barrier()

        @pl.when(s < 15)
        def _():
            pltpu.sync_copy(acc_sh.at[pl.ds(s * span, span), :],
                            out_hbm.at[pl.ds(lo + s * span, span), :])

        @pl.when(s == 15)
        def _():
            pltpu.sync_copy(acc_sh.at[pl.ds(15 * span, last), :],
                            out_hbm.at[pl.ds(lo + 15 * span, last), :])

    return gss


_a2c_sc = _make_gss(N, NC, SP_NC)
_c2a_sc = _make_gss(NC, N, SP_N)


_BN_EPS = 1e-5
_F32 = jnp.float32


def _dot(a, b):
    return jnp.dot(a, b, preferred_element_type=_F32)


def _stats_update(sums_ref, z, i):
    @pl.when(i == 0)
    def _():
        sums_ref[...] = jnp.zeros_like(sums_ref)

    sums_ref[0:1, :] += jnp.sum(z, axis=0, keepdims=True)
    sums_ref[1:2, :] += jnp.sum(z * z, axis=0, keepdims=True)


def _bn_apply(z, sums, nrows, g, b):
    m = sums[0:1, :] / nrows
    var = sums[1:2, :] / nrows - m * m
    return (z - m) * lax.rsqrt(var + _BN_EPS) * g + b


def _tk1_body(h_ref, aggr_ref, eps_ref, w1_ref, b1_ref, z1_ref, sums_ref):
    i = pl.program_id(0)
    u = (1.0 + eps_ref[0, 0]) * h_ref[...] + aggr_ref[...]
    z = _dot(u, w1_ref[...]) + b1_ref[...]
    z1_ref[...] = z
    _stats_update(sums_ref, z, i)


def _tk2_body(z1_ref, sums1_ref, g1_ref, bb1_ref, w2_ref, b2_ref,
              z2_ref, sums_ref):
    i = pl.program_id(0)
    v = jax.nn.relu(_bn_apply(z1_ref[...], sums1_ref[...], float(N),
                              g1_ref[...], bb1_ref[...]))
    z = _dot(v, w2_ref[...]) + b2_ref[...]
    z2_ref[...] = z
    _stats_update(sums_ref, z, i)


def _tk3_body(z2_ref, sums2_ref, g_ref, b_ref, h_ref):
    h_ref[...] = jax.nn.relu(_bn_apply(z2_ref[...], sums2_ref[...], float(N),
                                       g_ref[...], b_ref[...]))


def _tk4_body(cm_ref, cnt_ref, hc_ref, w_ref, b_ref, out_ref, sums_ref):
    i = pl.program_id(0)
    cm = cm_ref[...] / jnp.maximum(cnt_ref[:, 0:1], 1.0)
    z = hc_ref[...] + jax.nn.relu(_dot(cm, w_ref[...]) + b_ref[...])
    out_ref[...] = z
    _stats_update(sums_ref, z, i)


def _tk5_body(zp_ref, sums_ref, g_ref, b_ref, out_ref):
    out_ref[...] = jax.nn.relu(_bn_apply(zp_ref[...], sums_ref[...], float(NC),
                                         g_ref[...], b_ref[...]))


def _tk6_body(am_ref, cnt_ref, h_ref, w_ref, b_ref, out_ref):
    am = am_ref[...] / jnp.maximum(cnt_ref[:, 0:1], 1.0)
    out_ref[...] = h_ref[...] + jax.nn.relu(_dot(am, w_ref[...]) + b_ref[...])


def _row_spec(bs, cols):
    return pl.BlockSpec((bs, cols), lambda i: (i, 0))


def _full_spec(shape):
    return pl.BlockSpec(shape, lambda i: tuple(0 for _ in shape))


_BN_ROWS = 2000
_BC_ROWS = 1000


def _tc_call(body, grid, in_specs, out_specs, out_shapes):
    return pl.pallas_call(
        body, grid=(grid,), in_specs=in_specs, out_specs=out_specs,
        out_shape=out_shapes)


def _tc_call1(*args):
    def run(*ins):
        (out,) = _tc_call(*args)(*ins)
        return out
    return run


def _run_tk1(h, aggr, eps_l, w1, b1):
    return _tc_call(
        _tk1_body, N // _BN_ROWS,
        [_row_spec(_BN_ROWS, H), _row_spec(_BN_ROWS, H), _full_spec((8, 128)),
         _full_spec((H, 2 * H)), _full_spec((1, 2 * H))],
        [_row_spec(_BN_ROWS, 2 * H), _full_spec((8, 2 * H))],
        [jax.ShapeDtypeStruct((N, 2 * H), _F32),
         jax.ShapeDtypeStruct((8, 2 * H), _F32)],
    )(h, aggr, eps_l, w1, b1)


def _run_tk2(z1, sums1, g1, bb1, w2, b2):
    return _tc_call(
        _tk2_body, N // _BN_ROWS,
        [_row_spec(_BN_ROWS, 2 * H), _full_spec((8, 2 * H)),
         _full_spec((1, 2 * H)), _full_spec((1, 2 * H)),
         _full_spec((2 * H, H)), _full_spec((1, H))],
        [_row_spec(_BN_ROWS, H), _full_spec((8, H))],
        [jax.ShapeDtypeStruct((N, H), _F32),
         jax.ShapeDtypeStruct((8, H), _F32)],
    )(z1, sums1, g1, bb1, w2, b2)


def _run_tk3(z2, sums2, g, b):
    return _tc_call1(
        _tk3_body, N // _BN_ROWS,
        [_row_spec(_BN_ROWS, H), _full_spec((8, H)), _full_spec((1, H)),
         _full_spec((1, H))],
        [_row_spec(_BN_ROWS, H)],
        [jax.ShapeDtypeStruct((N, H), _F32)],
    )(z2, sums2, g, b)


def _run_tk4(cm_sum, cnt_c, hc, w, b):
    return _tc_call(
        _tk4_body, NC // _BC_ROWS,
        [_row_spec(_BC_ROWS, H), _row_spec(_BC_ROWS, 16), _row_spec(_BC_ROWS, H),
         _full_spec((H, H)), _full_spec((1, H))],
        [_row_spec(_BC_ROWS, H), _full_spec((8, H))],
        [jax.ShapeDtypeStruct((NC, H), _F32),
         jax.ShapeDtypeStruct((8, H), _F32)],
    )(cm_sum, cnt_c, hc, w, b)


def _run_tk5(hc_pre, sums_s, g, b):
    return _tc_call1(
        _tk5_body, NC // _BC_ROWS,
        [_row_spec(_BC_ROWS, H), _full_spec((8, H)), _full_spec((1, H)),
         _full_spec((1, H))],
        [_row_spec(_BC_ROWS, H)],
        [jax.ShapeDtypeStruct((NC, H), _F32)],
    )(hc_pre, sums_s, g, b)


def _run_tk6(am_sum, cnt_a, h_mid, w, b):
    return _tc_call1(
        _tk6_body, N // _BN_ROWS,
        [_row_spec(_BN_ROWS, H), _row_spec(_BN_ROWS, 16), _row_spec(_BN_ROWS, H),
         _full_spec((H, H)), _full_spec((1, H))],
        [_row_spec(_BN_ROWS, H)],
        [jax.ShapeDtypeStruct((N, H), _F32)],
    )(am_sum, cnt_a, h_mid, w, b)


def kernel(x, x_clique, graph_lpe, edge_index_graph, edge_attr_graph,
           atom2clique_row, atom2clique_col,
           atom_emb, clique_emb, clique_W, clique_b, bond_emb, eps,
           W1, b1, bn1_g, bn1_b, W2, b2, gn_g, gn_b, sn_g, sn_b,
           a2c_W, a2c_b, c2a_W, c2a_b):
    i32 = jnp.int32
    f32 = jnp.float32

    x = x.astype(i32)
    t9 = atom_emb.reshape(9 * 100, H).astype(f32)
    idx9 = (x + 100 * jnp.arange(9, dtype=i32)[None, :]).T
    idx9 = jnp.pad(idx9, ((0, 0), (0, N_PAD - N))).reshape(9 * N_PAD)

    t4 = (clique_emb @ clique_W + clique_b).astype(f32)
    xc = jnp.pad(x_clique.astype(i32), (0, NC_PAD - NC))

    src = edge_index_graph[0].astype(i32)
    dst = edge_index_graph[1].astype(i32)
    ea = edge_attr_graph.astype(i32)
    ce = ea[:, 0] * 36 + ea[:, 1] * 6 + ea[:, 2]
    src_p = jnp.pad(src, (0, E_PAD - E))
    ce_p = jnp.pad(ce, (0, E_PAD - E))
    dst_p = jnp.pad(dst, (0, E_PAD - E), constant_values=-1)

    row = atom2clique_row.astype(i32)
    col = atom2clique_col.astype(i32)
    row_g = jnp.pad(row, (0, A_PAD - A))
    col_g = jnp.pad(col, (0, A_PAD - A))
    row_s = jnp.pad(row, (0, A_PAD - A), constant_values=-1)
    col_s = jnp.pad(col, (0, A_PAD - A), constant_values=-1)

    bts = [
        (bond_emb[l, 0][:, None, None, :] + bond_emb[l, 1][None, :, None, :]
         + bond_emb[l, 2][None, None, :, :]).reshape(216, H).astype(f32)
        for l in range(3)
    ]

    zeros_n = jnp.zeros((1568, H), f32)
    zeros_cnt = jnp.zeros((3128, 16), f32)
    ones16 = jnp.ones((CH, 16), f32)

    h0_pad, hc0_pad = _encode_sc(t9, idx9, t4, xc)
    h = h0_pad[:N]
    hc = hc0_pad[:NC]
    cnt_a, cnt_c = _counts_sc(row_s, col_s, ones16, zeros_cnt)

    for l in range(3):
        eps_l = jnp.full((8, 128), eps[l], f32)
        aggr = _edge_sc(h, bts[l], src_p, ce_p, dst_p, zeros_n)
        z1, sums1 = _run_tk1(h, aggr, eps_l, W1[l],
                             b1[l].reshape(1, 2 * H))
        z2, sums2 = _run_tk2(z1, sums1, bn1_g[l].reshape(1, 2 * H),
                             bn1_b[l].reshape(1, 2 * H), W2[l],
                             b2[l].reshape(1, H))
        h_mid = _run_tk3(z2, sums2, gn_g[l].reshape(1, H),
                         gn_b[l].reshape(1, H))
        cm_sum = _a2c_sc(h_mid, row_g, col_s, zeros_n)
        hc_pre, sums_s = _run_tk4(cm_sum, cnt_c, hc, a2c_W[l],
                                  a2c_b[l].reshape(1, H))
        hc = _run_tk5(hc_pre, sums_s, sn_g[l].reshape(1, H),
                      sn_b[l].reshape(1, H))
        am_sum = _c2a_sc(hc, col_g, row_s, zeros_n)
        h = _run_tk6(am_sum, cnt_a, h_mid, c2a_W[l],
                     c2a_b[l].reshape(1, H))

    return h


# speedup vs baseline: 4.6968x; 1.0581x over previous
import functools

import jax
import jax.numpy as jnp
from jax import lax
from jax.experimental import pallas as pl
from jax.experimental.pallas import tpu as pltpu
from jax.experimental.pallas import tpu_sc as plsc

N = 50000
NC = 25000
E = 800000
A = 50000
H = 64

NCORE = 2
NSUB = 16
CH = 128

E_PAD = 800256
CE = 96
A_PAD = 51200
N_PAD = 50048
NC_PAD = 25088

HALF_N = N // 2
HALF_NC = NC // 2
SP_N = 25088
SP_NC = 12544

_MESH = plsc.VectorSubcoreMesh(core_axis_name="c", subcore_axis_name="s")
_SC_PARAMS = pltpu.CompilerParams(use_tc_tiling_on_sc=False)


def _relu_add_inplace(abuf, bbuf, rows):
    @pl.loop(0, rows, unroll=8)
    def _(r):
        for q in range(H // 16):
            sl = pl.ds(q * 16, 16)
            abuf[r, sl] = jnp.maximum(abuf[r, sl] + bbuf[r, sl], 0.0)


def _local_idx(dv, lo, half):
    valid = (dv >= lo) & (dv < lo + half)
    return jnp.where(valid, dv - lo, -1)


@functools.partial(
    pl.kernel,
    out_type=(
        jax.ShapeDtypeStruct((N_PAD, H), jnp.float32),
        jax.ShapeDtypeStruct((NC_PAD, H), jnp.float32),
    ),
    mesh=_MESH,
    compiler_params=_SC_PARAMS,
    scratch_types=[
        pltpu.VMEM((CH,), jnp.int32),
        pltpu.VMEM((CH, H), jnp.float32),
        pltpu.VMEM((CH, H), jnp.float32),
    ],
)
def _encode_sc(t9_hbm, idx9_hbm, t4_hbm, xc_hbm, h0_out, hc0_out,
               idxv, acc, tmp):
    w = lax.axis_index("s") * NCORE + lax.axis_index("c")

    n_chunks = N_PAD // CH

    @pl.loop(0, 13)
    def _(jj):
        j = w * 13 + jj

        @pl.when(j < n_chunks)
        def _():
            b = j * CH
            pltpu.sync_copy(idx9_hbm.at[pl.ds(b, CH)], idxv)
            pltpu.sync_copy(t9_hbm.at[idxv], acc)
            for i in range(1, 9):
                pltpu.sync_copy(idx9_hbm.at[pl.ds(i * N_PAD + b, CH)], idxv)
                pltpu.sync_copy(t9_hbm.at[idxv], tmp)

                @pl.loop(0, CH, unroll=8)
                def _(r):
                    for q in range(H // 16):
                        sl = pl.ds(q * 16, 16)
                        acc[r, sl] = acc[r, sl] + tmp[r, sl]
            pltpu.sync_copy(acc, h0_out.at[pl.ds(b, CH), :])

    c_chunks = NC_PAD // CH

    @pl.loop(0, 7)
    def _(jj):
        j = w * 7 + jj

        @pl.when(j < c_chunks)
        def _():
            b = j * CH
            pltpu.sync_copy(xc_hbm.at[pl.ds(b, CH)], idxv)
            pltpu.sync_copy(t4_hbm.at[idxv], acc)
            pltpu.sync_copy(acc, hc0_out.at[pl.ds(b, CH), :])


@functools.partial(
    pl.kernel,
    out_type=(
        jax.ShapeDtypeStruct((N, 16), jnp.float32),
        jax.ShapeDtypeStruct((NC, 16), jnp.float32),
    ),
    mesh=_MESH,
    compiler_params=_SC_PARAMS,
    scratch_types=[
        pltpu.VMEM((CH,), jnp.int32),
        pltpu.VMEM((CH, 16), jnp.float32),
        pltpu.VMEM_SHARED((N_PAD, 16), jnp.float32),
    ],
)
def _counts_sc(row_s_hbm, col_s_hbm, ones_hbm, zc_hbm, cnt_a_out, cnt_c_out,
               idxv, ones_v, cnt_sh):
    c = lax.axis_index("c")
    s = lax.axis_index("s")
    pltpu.sync_copy(ones_hbm, ones_v)

    @pl.when(c == 0)
    def _():
        pltpu.sync_copy(zc_hbm, cnt_sh.at[pl.ds(s * 3128, 3128), :])

    @pl.when(c == 1)
    def _():
        pltpu.sync_copy(zc_hbm.at[pl.ds(0, 1564), :],
                        cnt_sh.at[pl.ds(s * 1564, 1564), :])

    plsc.subcore_barrier()

    n_chunks = A_PAD // (NSUB * CH)

    @pl.loop(0, n_chunks)
    def _(j):
        b = s * (A_PAD // NSUB) + j * CH

        @pl.when(c == 0)
        def _():
            pltpu.sync_copy(row_s_hbm.at[pl.ds(b, CH)], idxv)
            pltpu.sync_copy(
                ones_v, cnt_sh.at[plsc.Indices(idxv, ignored_value=-1)],
                add=True)

        @pl.when(c == 1)
        def _():
            pltpu.sync_copy(col_s_hbm.at[pl.ds(b, CH)], idxv)
            pltpu.sync_copy(
                ones_v, cnt_sh.at[plsc.Indices(idxv, ignored_value=-1)],
                add=True)

    plsc.subcore_barrier()

    @pl.when(c == 0)
    def _():
        @pl.when(s < 15)
        def _():
            pltpu.sync_copy(cnt_sh.at[pl.ds(s * 3128, 3128), :],
                            cnt_a_out.at[pl.ds(s * 3128, 3128), :])

        @pl.when(s == 15)
        def _():
            pltpu.sync_copy(cnt_sh.at[pl.ds(15 * 3128, 3080), :],
                            cnt_a_out.at[pl.ds(15 * 3128, 3080), :])

    @pl.when(c == 1)
    def _():
        @pl.when(s < 15)
        def _():
            pltpu.sync_copy(cnt_sh.at[pl.ds(s * 1564, 1564), :],
                            cnt_c_out.at[pl.ds(s * 1564, 1564), :])

        @pl.when(s == 15)
        def _():
            pltpu.sync_copy(cnt_sh.at[pl.ds(15 * 1564, 1540), :],
                            cnt_c_out.at[pl.ds(15 * 1564, 1540), :])


@functools.partial(
    pl.kernel,
    out_type=jax.ShapeDtypeStruct((N, H), jnp.float32),
    mesh=_MESH,
    compiler_params=_SC_PARAMS,
    scratch_types=[
        [pltpu.VMEM((CE,), jnp.int32)] * 2,
        [pltpu.VMEM((CE,), jnp.int32)] * 2,
        [pltpu.VMEM((CE,), jnp.int32)] * 2,
        [pltpu.VMEM((CE,), jnp.int32)] * 2,
        [pltpu.VMEM((CE, H), jnp.float32)] * 2,
        [pltpu.VMEM((CE, H), jnp.float32)] * 2,
        [pltpu.SemaphoreType.DMA] * 2,
        [pltpu.SemaphoreType.DMA] * 2,
        [pltpu.SemaphoreType.DMA] * 2,
        pltpu.VMEM_SHARED((SP_N, H), jnp.float32),
    ],
)
def _edge_sc(h_hbm, bt_hbm, src_hbm, ce_hbm, dst_hbm, z_hbm, aggr_out,
             srcv, cev, dstv, dlv, hbuf, ebuf, sem_idx, sem_g, sem_s,
             aggr_sh):
    c = lax.axis_index("c")
    s = lax.axis_index("s")
    lo = c * HALF_N

    pltpu.sync_copy(z_hbm.at[pl.ds(0, 1568), :],
                    aggr_sh.at[pl.ds(s * 1568, 1568), :])
    plsc.subcore_barrier()

    per_sub = E_PAD // NSUB
    n_chunks = per_sub // CE

    def fire_idx(b, j):
        bb = s * per_sub + j * CE
        pltpu.async_copy(src_hbm.at[pl.ds(bb, CE)], srcv[b], sem_idx[b])
        pltpu.async_copy(ce_hbm.at[pl.ds(bb, CE)], cev[b], sem_idx[b])
        pltpu.async_copy(dst_hbm.at[pl.ds(bb, CE)], dstv[b], sem_idx[b])

    def wait_idx(b):
        pltpu.make_async_copy(src_hbm.at[pl.ds(0, CE)], srcv[b],
                              sem_idx[b]).wait()
        pltpu.make_async_copy(ce_hbm.at[pl.ds(0, CE)], cev[b],
                              sem_idx[b]).wait()
        pltpu.make_async_copy(dst_hbm.at[pl.ds(0, CE)], dstv[b],
                              sem_idx[b]).wait()
        for q in range(CE // 16):
            sl = pl.ds(q * 16, 16)
            dl = _local_idx(dstv[b][sl], lo, HALF_N)
            dlv[b][sl] = dl
            srcv[b][sl] = jnp.where(dl >= 0, srcv[b][sl], -1)
            cev[b][sl] = jnp.where(dl >= 0, cev[b][sl], -1)

    def fire_gathers(b):
        pltpu.async_copy(h_hbm.at[plsc.Indices(srcv[b], ignored_value=-1)],
                         hbuf[b], sem_g[b])
        pltpu.async_copy(bt_hbm.at[plsc.Indices(cev[b], ignored_value=-1)],
                         ebuf[b], sem_g[b])

    def wait_gathers(b):
        pltpu.make_async_copy(
            h_hbm.at[plsc.Indices(srcv[b], ignored_value=-1)], hbuf[b],
            sem_g[b]).wait()
        pltpu.make_async_copy(
            bt_hbm.at[plsc.Indices(cev[b], ignored_value=-1)], ebuf[b],
            sem_g[b]).wait()

    def fire_scatter(b):
        _relu_add_inplace(hbuf[b], ebuf[b], CE)
        pltpu.async_copy(
            hbuf[b], aggr_sh.at[plsc.Indices(dlv[b], ignored_value=-1)],
            sem_s[b], add=True)

    def wait_scatter(b):
        pltpu.make_async_copy(
            hbuf[b], aggr_sh.at[plsc.Indices(dlv[b], ignored_value=-1)],
            sem_s[b]).wait()

    fire_idx(0, 0)
    wait_idx(0)
    fire_gathers(0)
    fire_idx(1, 1)

    @pl.loop(0, n_chunks)
    def _(j):
        def body(b, nb):
            @pl.when(j + 1 < n_chunks)
            def _():
                @pl.when(j >= 1)
                def _():
                    wait_scatter(nb)

                wait_idx(nb)
                fire_gathers(nb)

            wait_gathers(b)
            fire_scatter(b)

            @pl.when(j + 2 < n_chunks)
            def _():
                fire_idx(b, j + 2)

        @pl.when(j % 2 == 0)
        def _():
            body(0, 1)

        @pl.when(j % 2 == 1)
        def _():
            body(1, 0)

    wait_scatter(0)
    wait_scatter(1)
    plsc.subcore_barrier()

    @pl.when(s < 15)
    def _():
        pltpu.sync_copy(aggr_sh.at[pl.ds(s * 1568, 1568), :],
                        aggr_out.at[pl.ds(lo + s * 1568, 1568), :])

    @pl.when(s == 15)
    def _():
        pltpu.sync_copy(aggr_sh.at[pl.ds(15 * 1568, 1480), :],
                        aggr_out.at[pl.ds(lo + 15 * 1568, 1480), :])


def _make_gss(table_rows, out_rows, sp_rows):
    half = out_rows // 2
    span = sp_rows // NSUB
    last = half - 15 * span
    n_chunks = A_PAD // (NSUB * CH)

    @functools.partial(
        pl.kernel,
        out_type=jax.ShapeDtypeStruct((out_rows, H), jnp.float32),
        mesh=_MESH,
        compiler_params=_SC_PARAMS,
        scratch_types=[
            pltpu.VMEM((CH,), jnp.int32),
            pltpu.VMEM((CH,), jnp.int32),
            pltpu.VMEM((CH,), jnp.int32),
            pltpu.VMEM((CH, H), jnp.float32),
            pltpu.VMEM_SHARED((sp_rows, H), jnp.float32),
        ],
    )
    def gss(table_hbm, gidx_hbm, sidx_hbm, z_hbm, out_hbm,
            gv, sv, dlv, buf, acc_sh):
        c = lax.axis_index("c")
        s = lax.axis_index("s")
        lo = c * half

        pltpu.sync_copy(z_hbm.at[pl.ds(0, span), :],
                        acc_sh.at[pl.ds(s * span, span), :])
        plsc.subcore_barrier()

        @pl.loop(0, n_chunks)
        def _(j):
            b = s * (A_PAD // NSUB) + j * CH
            pltpu.sync_copy(gidx_hbm.at[pl.ds(b, CH)], gv)
            pltpu.sync_copy(sidx_hbm.at[pl.ds(b, CH)], sv)
            pltpu.sync_copy(table_hbm.at[gv], buf)
            for k in range(CH // 16):
                sl = pl.ds(k * 16, 16)
                dlv[sl] = _local_idx(sv[sl], lo, half)
            pltpu.sync_copy(
                buf, acc_sh.at[plsc.Indices(dlv, ignored_value=-1)], add=True)

        plsc.subcore_barrier()

        @pl.when(s < 15)
        def _():
            pltpu.sync_copy(acc_sh.at[pl.ds(s * span, span), :],
                            out_hbm.at[pl.ds(lo + s * span, span), :])

        @pl.when(s == 15)
        def _():
            pltpu.sync_copy(acc_sh.at[pl.ds(15 * span, last), :],
                            out_hbm.at[pl.ds(lo + 15 * span, last), :])

    return gss


_a2c_sc = _make_gss(N, NC, SP_NC)
_c2a_sc = _make_gss(NC, N, SP_N)


_BN_EPS = 1e-5
_F32 = jnp.float32


def _dot(a, b):
    return jnp.dot(a, b, preferred_element_type=_F32)


def _stats_update(sums_ref, z, i):
    @pl.when(i == 0)
    def _():
        sums_ref[...] = jnp.zeros_like(sums_ref)

    sums_ref[0:1, :] += jnp.sum(z, axis=0, keepdims=True)
    sums_ref[1:2, :] += jnp.sum(z * z, axis=0, keepdims=True)


def _bn_apply(z, sums, nrows, g, b):
    m = sums[0:1, :] / nrows
    var = sums[1:2, :] / nrows - m * m
    return (z - m) * lax.rsqrt(var + _BN_EPS) * g + b


def _tk1_body(h_ref, aggr_ref, eps_ref, w1_ref, b1_ref, z1_ref, sums_ref):
    i = pl.program_id(0)
    u = (1.0 + eps_ref[0, 0]) * h_ref[...] + aggr_ref[...]
    z = _dot(u, w1_ref[...]) + b1_ref[...]
    z1_ref[...] = z
    _stats_update(sums_ref, z, i)


def _tk2_body(z1_ref, sums1_ref, g1_ref, bb1_ref, w2_ref, b2_ref,
              z2_ref, sums_ref):
    i = pl.program_id(0)
    v = jax.nn.relu(_bn_apply(z1_ref[...], sums1_ref[...], float(N),
                              g1_ref[...], bb1_ref[...]))
    z = _dot(v, w2_ref[...]) + b2_ref[...]
    z2_ref[...] = z
    _stats_update(sums_ref, z, i)


def _tk3_body(z2_ref, sums2_ref, g_ref, b_ref, h_ref):
    h_ref[...] = jax.nn.relu(_bn_apply(z2_ref[...], sums2_ref[...], float(N),
                                       g_ref[...], b_ref[...]))


def _tk4_body(cm_ref, cnt_ref, hc_ref, w_ref, b_ref, out_ref, sums_ref):
    i = pl.program_id(0)
    cm = cm_ref[...] / jnp.maximum(cnt_ref[:, 0:1], 1.0)
    z = hc_ref[...] + jax.nn.relu(_dot(cm, w_ref[...]) + b_ref[...])
    out_ref[...] = z
    _stats_update(sums_ref, z, i)


def _tk5_body(zp_ref, sums_ref, g_ref, b_ref, out_ref):
    out_ref[...] = jax.nn.relu(_bn_apply(zp_ref[...], sums_ref[...], float(NC),
                                         g_ref[...], b_ref[...]))


def _tk6_body(am_ref, cnt_ref, h_ref, w_ref, b_ref, out_ref):
    am = am_ref[...] / jnp.maximum(cnt_ref[:, 0:1], 1.0)
    out_ref[...] = h_ref[...] + jax.nn.relu(_dot(am, w_ref[...]) + b_ref[...])


def _row_spec(bs, cols):
    return pl.BlockSpec((bs, cols), lambda i: (i, 0))


def _full_spec(shape):
    return pl.BlockSpec(shape, lambda i: tuple(0 for _ in shape))


_BN_ROWS = 2000
_BC_ROWS = 1000


def _tc_call(body, grid, in_specs, out_specs, out_shapes):
    return pl.pallas_call(
        body, grid=(grid,), in_specs=in_specs, out_specs=out_specs,
        out_shape=out_shapes)


def _tc_call1(*args):
    def run(*ins):
        (out,) = _tc_call(*args)(*ins)
        return out
    return run


def _run_tk1(h, aggr, eps_l, w1, b1):
    return _tc_call(
        _tk1_body, N // _BN_ROWS,
        [_row_spec(_BN_ROWS, H), _row_spec(_BN_ROWS, H), _full_spec((8, 128)),
         _full_spec((H, 2 * H)), _full_spec((1, 2 * H))],
        [_row_spec(_BN_ROWS, 2 * H), _full_spec((8, 2 * H))],
        [jax.ShapeDtypeStruct((N, 2 * H), _F32),
         jax.ShapeDtypeStruct((8, 2 * H), _F32)],
    )(h, aggr, eps_l, w1, b1)


def _run_tk2(z1, sums1, g1, bb1, w2, b2):
    return _tc_call(
        _tk2_body, N // _BN_ROWS,
        [_row_spec(_BN_ROWS, 2 * H), _full_spec((8, 2 * H)),
         _full_spec((1, 2 * H)), _full_spec((1, 2 * H)),
         _full_spec((2 * H, H)), _full_spec((1, H))],
        [_row_spec(_BN_ROWS, H), _full_spec((8, H))],
        [jax.ShapeDtypeStruct((N, H), _F32),
         jax.ShapeDtypeStruct((8, H), _F32)],
    )(z1, sums1, g1, bb1, w2, b2)


def _run_tk3(z2, sums2, g, b):
    return _tc_call1(
        _tk3_body, N // _BN_ROWS,
        [_row_spec(_BN_ROWS, H), _full_spec((8, H)), _full_spec((1, H)),
         _full_spec((1, H))],
        [_row_spec(_BN_ROWS, H)],
        [jax.ShapeDtypeStruct((N, H), _F32)],
    )(z2, sums2, g, b)


def _run_tk4(cm_sum, cnt_c, hc, w, b):
    return _tc_call(
        _tk4_body, NC // _BC_ROWS,
        [_row_spec(_BC_ROWS, H), _row_spec(_BC_ROWS, 16), _row_spec(_BC_ROWS, H),
         _full_spec((H, H)), _full_spec((1, H))],
        [_row_spec(_BC_ROWS, H), _full_spec((8, H))],
        [jax.ShapeDtypeStruct((NC, H), _F32),
         jax.ShapeDtypeStruct((8, H), _F32)],
    )(cm_sum, cnt_c, hc, w, b)


def _run_tk5(hc_pre, sums_s, g, b):
    return _tc_call1(
        _tk5_body, NC // _BC_ROWS,
        [_row_spec(_BC_ROWS, H), _full_spec((8, H)), _full_spec((1, H)),
         _full_spec((1, H))],
        [_row_spec(_BC_ROWS, H)],
        [jax.ShapeDtypeStruct((NC, H), _F32)],
    )(hc_pre, sums_s, g, b)


def _run_tk6(am_sum, cnt_a, h_mid, w, b):
    return _tc_call1(
        _tk6_body, N // _BN_ROWS,
        [_row_spec(_BN_ROWS, H), _row_spec(_BN_ROWS, 16), _row_spec(_BN_ROWS, H),
         _full_spec((H, H)), _full_spec((1, H))],
        [_row_spec(_BN_ROWS, H)],
        [jax.ShapeDtypeStruct((N, H), _F32)],
    )(am_sum, cnt_a, h_mid, w, b)


def kernel(x, x_clique, graph_lpe, edge_index_graph, edge_attr_graph,
           atom2clique_row, atom2clique_col,
           atom_emb, clique_emb, clique_W, clique_b, bond_emb, eps,
           W1, b1, bn1_g, bn1_b, W2, b2, gn_g, gn_b, sn_g, sn_b,
           a2c_W, a2c_b, c2a_W, c2a_b):
    i32 = jnp.int32
    f32 = jnp.float32

    x = x.astype(i32)
    t9 = atom_emb.reshape(9 * 100, H).astype(f32)
    idx9 = (x + 100 * jnp.arange(9, dtype=i32)[None, :]).T
    idx9 = jnp.pad(idx9, ((0, 0), (0, N_PAD - N))).reshape(9 * N_PAD)

    t4 = (clique_emb @ clique_W + clique_b).astype(f32)
    xc = jnp.pad(x_clique.astype(i32), (0, NC_PAD - NC))

    src = edge_index_graph[0].astype(i32)
    dst = edge_index_graph[1].astype(i32)
    ea = edge_attr_graph.astype(i32)
    ce = ea[:, 0] * 36 + ea[:, 1] * 6 + ea[:, 2]
    src_p = jnp.pad(src, (0, E_PAD - E))
    ce_p = jnp.pad(ce, (0, E_PAD - E))
    dst_p = jnp.pad(dst, (0, E_PAD - E), constant_values=-1)

    row = atom2clique_row.astype(i32)
    col = atom2clique_col.astype(i32)
    row_g = jnp.pad(row, (0, A_PAD - A))
    col_g = jnp.pad(col, (0, A_PAD - A))
    row_s = jnp.pad(row, (0, A_PAD - A), constant_values=-1)
    col_s = jnp.pad(col, (0, A_PAD - A), constant_values=-1)

    bts = [
        (bond_emb[l, 0][:, None, None, :] + bond_emb[l, 1][None, :, None, :]
         + bond_emb[l, 2][None, None, :, :]).reshape(216, H).astype(f32)
        for l in range(3)
    ]

    zeros_n = jnp.zeros((1568, H), f32)
    zeros_cnt = jnp.zeros((3128, 16), f32)
    ones16 = jnp.ones((CH, 16), f32)

    h0_pad, hc0_pad = _encode_sc(t9, idx9, t4, xc)
    h = h0_pad[:N]
    hc = hc0_pad[:NC]
    cnt_a, cnt_c = _counts_sc(row_s, col_s, ones16, zeros_cnt)

    for l in range(3):
        eps_l = jnp.full((8, 128), eps[l], f32)
        aggr = _edge_sc(h, bts[l], src_p, ce_p, dst_p, zeros_n)
        z1, sums1 = _run_tk1(h, aggr, eps_l, W1[l],
                             b1[l].reshape(1, 2 * H))
        z2, sums2 = _run_tk2(z1, sums1, bn1_g[l].reshape(1, 2 * H),
                             bn1_b[l].reshape(1, 2 * H), W2[l],
                             b2[l].reshape(1, H))
        h_mid = _run_tk3(z2, sums2, gn_g[l].reshape(1, H),
                         gn_b[l].reshape(1, H))
        cm_sum = _a2c_sc(h_mid, row_g, col_s, zeros_n)
        hc_pre, sums_s = _run_tk4(cm_sum, cnt_c, hc, a2c_W[l],
                                  a2c_b[l].reshape(1, H))
        hc = _run_tk5(hc_pre, sums_s, sn_g[l].reshape(1, H),
                      sn_b[l].reshape(1, H))
        am_sum = _c2a_sc(hc, col_g, row_s, zeros_n)
        h = _run_tk6(am_sum, cnt_a, h_mid, c2a_W[l],
                     c2a_b[l].reshape(1, H))

    return h


# speedup vs baseline: 5.9111x; 1.2585x over previous
import functools

import jax
import jax.numpy as jnp
from jax import lax
from jax.experimental import pallas as pl
from jax.experimental.pallas import tpu as pltpu
from jax.experimental.pallas import tpu_sc as plsc

N = 50000
NC = 25000
E = 800000
A = 50000
H = 64

NCORE = 2
NSUB = 16
CH = 128

E_PAD = 800768
CE = 128
A_PAD = 51200
N_PAD = 50048
NC_PAD = 25088

HALF_N = N // 2
HALF_NC = NC // 2
SP_N = 25088
SP_NC = 12544

_MESH = plsc.VectorSubcoreMesh(core_axis_name="c", subcore_axis_name="s")
_SC_PARAMS = pltpu.CompilerParams(use_tc_tiling_on_sc=False)


def _local_idx(dv, lo, half):
    valid = (dv >= lo) & (dv < lo + half)
    return jnp.where(valid, dv - lo, -1)


@functools.partial(
    pl.kernel,
    out_type=(
        jax.ShapeDtypeStruct((N_PAD, H), jnp.float32),
        jax.ShapeDtypeStruct((NC_PAD, H), jnp.float32),
    ),
    mesh=_MESH,
    compiler_params=_SC_PARAMS,
    scratch_types=[
        pltpu.VMEM((CH,), jnp.int32),
        pltpu.VMEM((CH, H), jnp.float32),
        pltpu.VMEM((CH, H), jnp.float32),
    ],
)
def _encode_sc(t9_hbm, idx9_hbm, t4_hbm, xc_hbm, h0_out, hc0_out,
               idxv, acc, tmp):
    w = lax.axis_index("s") * NCORE + lax.axis_index("c")

    n_chunks = N_PAD // CH

    @pl.loop(0, 13)
    def _(jj):
        j = w * 13 + jj

        @pl.when(j < n_chunks)
        def _():
            b = j * CH
            pltpu.sync_copy(idx9_hbm.at[pl.ds(b, CH)], idxv)
            pltpu.sync_copy(t9_hbm.at[idxv], acc)
            for i in range(1, 9):
                pltpu.sync_copy(idx9_hbm.at[pl.ds(i * N_PAD + b, CH)], idxv)
                pltpu.sync_copy(t9_hbm.at[idxv], tmp)

                @pl.loop(0, CH, unroll=8)
                def _(r):
                    for q in range(H // 16):
                        sl = pl.ds(q * 16, 16)
                        acc[r, sl] = acc[r, sl] + tmp[r, sl]
            pltpu.sync_copy(acc, h0_out.at[pl.ds(b, CH), :])

    c_chunks = NC_PAD // CH

    @pl.loop(0, 7)
    def _(jj):
        j = w * 7 + jj

        @pl.when(j < c_chunks)
        def _():
            b = j * CH
            pltpu.sync_copy(xc_hbm.at[pl.ds(b, CH)], idxv)
            pltpu.sync_copy(t4_hbm.at[idxv], acc)
            pltpu.sync_copy(acc, hc0_out.at[pl.ds(b, CH), :])


@functools.partial(
    pl.kernel,
    out_type=(
        jax.ShapeDtypeStruct((N, 16), jnp.float32),
        jax.ShapeDtypeStruct((NC, 16), jnp.float32),
    ),
    mesh=_MESH,
    compiler_params=_SC_PARAMS,
    scratch_types=[
        pltpu.VMEM((CH,), jnp.int32),
        pltpu.VMEM((CH, 16), jnp.float32),
        pltpu.VMEM_SHARED((N_PAD, 16), jnp.float32),
    ],
)
def _counts_sc(row_s_hbm, col_s_hbm, ones_hbm, zc_hbm, cnt_a_out, cnt_c_out,
               idxv, ones_v, cnt_sh):
    c = lax.axis_index("c")
    s = lax.axis_index("s")
    pltpu.sync_copy(ones_hbm, ones_v)

    @pl.when(c == 0)
    def _():
        pltpu.sync_copy(zc_hbm, cnt_sh.at[pl.ds(s * 3128, 3128), :])

    @pl.when(c == 1)
    def _():
        pltpu.sync_copy(zc_hbm.at[pl.ds(0, 1564), :],
                        cnt_sh.at[pl.ds(s * 1564, 1564), :])

    plsc.subcore_barrier()

    n_chunks = A_PAD // (NSUB * CH)

    @pl.loop(0, n_chunks)
    def _(j):
        b = s * (A_PAD // NSUB) + j * CH

        @pl.when(c == 0)
        def _():
            pltpu.sync_copy(row_s_hbm.at[pl.ds(b, CH)], idxv)
            pltpu.sync_copy(
                ones_v, cnt_sh.at[plsc.Indices(idxv, ignored_value=-1)],
                add=True)

        @pl.when(c == 1)
        def _():
            pltpu.sync_copy(col_s_hbm.at[pl.ds(b, CH)], idxv)
            pltpu.sync_copy(
                ones_v, cnt_sh.at[plsc.Indices(idxv, ignored_value=-1)],
                add=True)

    plsc.subcore_barrier()

    @pl.when(c == 0)
    def _():
        @pl.when(s < 15)
        def _():
            pltpu.sync_copy(cnt_sh.at[pl.ds(s * 3128, 3128), :],
                            cnt_a_out.at[pl.ds(s * 3128, 3128), :])

        @pl.when(s == 15)
        def _():
            pltpu.sync_copy(cnt_sh.at[pl.ds(15 * 3128, 3080), :],
                            cnt_a_out.at[pl.ds(15 * 3128, 3080), :])

    @pl.when(c == 1)
    def _():
        @pl.when(s < 15)
        def _():
            pltpu.sync_copy(cnt_sh.at[pl.ds(s * 1564, 1564), :],
                            cnt_c_out.at[pl.ds(s * 1564, 1564), :])

        @pl.when(s == 15)
        def _():
            pltpu.sync_copy(cnt_sh.at[pl.ds(15 * 1564, 1540), :],
                            cnt_c_out.at[pl.ds(15 * 1564, 1540), :])


@functools.partial(
    pl.kernel,
    out_type=jax.ShapeDtypeStruct((N, H), jnp.float32),
    mesh=_MESH,
    compiler_params=_SC_PARAMS,
    scratch_types=[
        [pltpu.VMEM((CE,), jnp.int32)] * 2,
        [pltpu.VMEM((CE,), jnp.int32)] * 2,
        [pltpu.VMEM((CE,), jnp.int32)] * 2,
        [pltpu.VMEM((CE,), jnp.int32)] * 2,
        [pltpu.VMEM((CE, H), jnp.float32)] * 2,
        [pltpu.SemaphoreType.DMA] * 2,
        [pltpu.SemaphoreType.DMA] * 2,
        [pltpu.SemaphoreType.DMA] * 2,
        pltpu.VMEM_SHARED((SP_N, H), jnp.float32),
    ],
)
def _edge_sc(h_hbm, bt_hbm, src_hbm, ce_hbm, dst_hbm, z_hbm, aggr_out,
             srcv, cev, dstv, dlv, hbuf, sem_idx, sem_g, sem_s,
             aggr_sh):
    c = lax.axis_index("c")
    s = lax.axis_index("s")
    lo = c * HALF_N

    pltpu.sync_copy(z_hbm.at[pl.ds(0, 1568), :],
                    aggr_sh.at[pl.ds(s * 1568, 1568), :])
    plsc.subcore_barrier()

    per_sub = E_PAD // NSUB
    n_chunks = per_sub // CE

    def fire_idx(b, j):
        bb = s * per_sub + j * CE
        pltpu.async_copy(src_hbm.at[pl.ds(bb, CE)], srcv[b], sem_idx[b])
        pltpu.async_copy(ce_hbm.at[pl.ds(bb, CE)], cev[b], sem_idx[b])
        pltpu.async_copy(dst_hbm.at[pl.ds(bb, CE)], dstv[b], sem_idx[b])

    def wait_idx(b):
        pltpu.make_async_copy(src_hbm.at[pl.ds(0, CE)], srcv[b],
                              sem_idx[b]).wait()
        pltpu.make_async_copy(ce_hbm.at[pl.ds(0, CE)], cev[b],
                              sem_idx[b]).wait()
        pltpu.make_async_copy(dst_hbm.at[pl.ds(0, CE)], dstv[b],
                              sem_idx[b]).wait()

    def fire_h(b):
        pltpu.async_copy(h_hbm.at[srcv[b]], hbuf[b], sem_g[b])

    def wait_h(b):
        pltpu.make_async_copy(h_hbm.at[srcv[b]], hbuf[b], sem_g[b]).wait()

    def fire_bt_add(b):
        pltpu.async_copy(bt_hbm.at[cev[b]], hbuf[b], sem_g[b], add=True)

    def wait_bt(b):
        pltpu.make_async_copy(bt_hbm.at[cev[b]], hbuf[b], sem_g[b]).wait()

    def fire_scatter(b):
        @pl.loop(0, CE, unroll=8)
        def _(r):
            for q in range(H // 16):
                sl = pl.ds(q * 16, 16)
                hbuf[b][r, sl] = jnp.maximum(hbuf[b][r, sl], 0.0)

        for q in range(CE // 16):
            sl = pl.ds(q * 16, 16)
            dlv[b][sl] = _local_idx(dstv[b][sl], lo, HALF_N)
        pltpu.async_copy(
            hbuf[b], aggr_sh.at[plsc.Indices(dlv[b], ignored_value=-1)],
            sem_s[b], add=True)

    def wait_scatter(b):
        pltpu.make_async_copy(
            hbuf[b], aggr_sh.at[plsc.Indices(dlv[b], ignored_value=-1)],
            sem_s[b]).wait()

    fire_idx(0, 0)
    wait_idx(0)
    fire_h(0)
    fire_idx(1, 1)

    @pl.loop(0, n_chunks)
    def _(j):
        def body(b, nb):
            wait_h(b)
            fire_bt_add(b)

            @pl.when(j + 1 < n_chunks)
            def _():
                @pl.when(j >= 1)
                def _():
                    wait_scatter(nb)

                wait_idx(nb)
                fire_h(nb)

            wait_bt(b)
            fire_scatter(b)

            @pl.when(j + 2 < n_chunks)
            def _():
                fire_idx(b, j + 2)

        @pl.when(j % 2 == 0)
        def _():
            body(0, 1)

        @pl.when(j % 2 == 1)
        def _():
            body(1, 0)

    wait_scatter(0)
    wait_scatter(1)
    plsc.subcore_barrier()

    @pl.when(s < 15)
    def _():
        pltpu.sync_copy(aggr_sh.at[pl.ds(s * 1568, 1568), :],
                        aggr_out.at[pl.ds(lo + s * 1568, 1568), :])

    @pl.when(s == 15)
    def _():
        pltpu.sync_copy(aggr_sh.at[pl.ds(15 * 1568, 1480), :],
                        aggr_out.at[pl.ds(lo + 15 * 1568, 1480), :])


def _make_gss(table_rows, out_rows, sp_rows):
    half = out_rows // 2
    span = sp_rows // NSUB
    last = half - 15 * span
    n_chunks = A_PAD // (NSUB * CH)

    @functools.partial(
        pl.kernel,
        out_type=jax.ShapeDtypeStruct((out_rows, H), jnp.float32),
        mesh=_MESH,
        compiler_params=_SC_PARAMS,
        scratch_types=[
            pltpu.VMEM((CH,), jnp.int32),
            pltpu.VMEM((CH,), jnp.int32),
            pltpu.VMEM((CH,), jnp.int32),
            pltpu.VMEM((CH, H), jnp.float32),
            pltpu.VMEM_SHARED((sp_rows, H), jnp.float32),
        ],
    )
    def gss(table_hbm, gidx_hbm, sidx_hbm, z_hbm, out_hbm,
            gv, sv, dlv, buf, acc_sh):
        c = lax.axis_index("c")
        s = lax.axis_index("s")
        lo = c * half

        pltpu.sync_copy(z_hbm.at[pl.ds(0, span), :],
                        acc_sh.at[pl.ds(s * span, span), :])
        plsc.subcore_barrier()

        @pl.loop(0, n_chunks)
        def _(j):
            b = s * (A_PAD // NSUB) + j * CH
            pltpu.sync_copy(gidx_hbm.at[pl.ds(b, CH)], gv)
            pltpu.sync_copy(sidx_hbm.at[pl.ds(b, CH)], sv)
            pltpu.sync_copy(table_hbm.at[gv], buf)
            for k in range(CH // 16):
                sl = pl.ds(k * 16, 16)
                dlv[sl] = _local_idx(sv[sl], lo, half)
            pltpu.sync_copy(
                buf, acc_sh.at[plsc.Indices(dlv, ignored_value=-1)], add=True)

        plsc.subcore_barrier()

        @pl.when(s < 15)
        def _():
            pltpu.sync_copy(acc_sh.at[pl.ds(s * span, span), :],
                            out_hbm.at[pl.ds(lo + s * span, span), :])

        @pl.when(s == 15)
        def _():
            pltpu.sync_copy(acc_sh.at[pl.ds(15 * span, last), :],
                            out_hbm.at[pl.ds(lo + 15 * span, last), :])

    return gss


_a2c_sc = _make_gss(N, NC, SP_NC)
_c2a_sc = _make_gss(NC, N, SP_N)


_BN_EPS = 1e-5
_F32 = jnp.float32


def _dot(a, b):
    return jnp.dot(a, b, preferred_element_type=_F32)


def _stats_update(sums_ref, z, i):
    @pl.when(i == 0)
    def _():
        sums_ref[...] = jnp.zeros_like(sums_ref)

    sums_ref[0:1, :] += jnp.sum(z, axis=0, keepdims=True)
    sums_ref[1:2, :] += jnp.sum(z * z, axis=0, keepdims=True)


def _bn_apply(z, sums, nrows, g, b):
    m = sums[0:1, :] / nrows
    var = sums[1:2, :] / nrows - m * m
    return (z - m) * lax.rsqrt(var + _BN_EPS) * g + b


def _tk1_body(h_ref, aggr_ref, eps_ref, w1_ref, b1_ref, z1_ref, sums_ref):
    i = pl.program_id(0)
    u = (1.0 + eps_ref[0, 0]) * h_ref[...] + aggr_ref[...]
    z = _dot(u, w1_ref[...]) + b1_ref[...]
    z1_ref[...] = z
    _stats_update(sums_ref, z, i)


def _tk2_body(z1_ref, sums1_ref, g1_ref, bb1_ref, w2_ref, b2_ref,
              z2_ref, sums_ref):
    i = pl.program_id(0)
    v = jax.nn.relu(_bn_apply(z1_ref[...], sums1_ref[...], float(N),
                              g1_ref[...], bb1_ref[...]))
    z = _dot(v, w2_ref[...]) + b2_ref[...]
    z2_ref[...] = z
    _stats_update(sums_ref, z, i)


def _tk3_body(z2_ref, sums2_ref, g_ref, b_ref, h_ref):
    h_ref[...] = jax.nn.relu(_bn_apply(z2_ref[...], sums2_ref[...], float(N),
                                       g_ref[...], b_ref[...]))


def _tk4_body(cm_ref, cnt_ref, hc_ref, w_ref, b_ref, out_ref, sums_ref):
    i = pl.program_id(0)
    cm = cm_ref[...] / jnp.maximum(cnt_ref[:, 0:1], 1.0)
    z = hc_ref[...] + jax.nn.relu(_dot(cm, w_ref[...]) + b_ref[...])
    out_ref[...] = z
    _stats_update(sums_ref, z, i)


def _tk5_body(zp_ref, sums_ref, g_ref, b_ref, out_ref):
    out_ref[...] = jax.nn.relu(_bn_apply(zp_ref[...], sums_ref[...], float(NC),
                                         g_ref[...], b_ref[...]))


def _tk6_body(am_ref, cnt_ref, h_ref, w_ref, b_ref, out_ref):
    am = am_ref[...] / jnp.maximum(cnt_ref[:, 0:1], 1.0)
    out_ref[...] = h_ref[...] + jax.nn.relu(_dot(am, w_ref[...]) + b_ref[...])


def _row_spec(bs, cols):
    return pl.BlockSpec((bs, cols), lambda i: (i, 0))


def _full_spec(shape):
    return pl.BlockSpec(shape, lambda i: tuple(0 for _ in shape))


_BN_ROWS = 2000
_BC_ROWS = 1000


def _tc_call(body, grid, in_specs, out_specs, out_shapes):
    return pl.pallas_call(
        body, grid=(grid,), in_specs=in_specs, out_specs=out_specs,
        out_shape=out_shapes)


def _tc_call1(*args):
    def run(*ins):
        (out,) = _tc_call(*args)(*ins)
        return out
    return run


def _run_tk1(h, aggr, eps_l, w1, b1):
    return _tc_call(
        _tk1_body, N // _BN_ROWS,
        [_row_spec(_BN_ROWS, H), _row_spec(_BN_ROWS, H), _full_spec((8, 128)),
         _full_spec((H, 2 * H)), _full_spec((1, 2 * H))],
        [_row_spec(_BN_ROWS, 2 * H), _full_spec((8, 2 * H))],
        [jax.ShapeDtypeStruct((N, 2 * H), _F32),
         jax.ShapeDtypeStruct((8, 2 * H), _F32)],
    )(h, aggr, eps_l, w1, b1)


def _run_tk2(z1, sums1, g1, bb1, w2, b2):
    return _tc_call(
        _tk2_body, N // _BN_ROWS,
        [_row_spec(_BN_ROWS, 2 * H), _full_spec((8, 2 * H)),
         _full_spec((1, 2 * H)), _full_spec((1, 2 * H)),
         _full_spec((2 * H, H)), _full_spec((1, H))],
        [_row_spec(_BN_ROWS, H), _full_spec((8, H))],
        [jax.ShapeDtypeStruct((N, H), _F32),
         jax.ShapeDtypeStruct((8, H), _F32)],
    )(z1, sums1, g1, bb1, w2, b2)


def _run_tk3(z2, sums2, g, b):
    return _tc_call1(
        _tk3_body, N // _BN_ROWS,
        [_row_spec(_BN_ROWS, H), _full_spec((8, H)), _full_spec((1, H)),
         _full_spec((1, H))],
        [_row_spec(_BN_ROWS, H)],
        [jax.ShapeDtypeStruct((N, H), _F32)],
    )(z2, sums2, g, b)


def _run_tk4(cm_sum, cnt_c, hc, w, b):
    return _tc_call(
        _tk4_body, NC // _BC_ROWS,
        [_row_spec(_BC_ROWS, H), _row_spec(_BC_ROWS, 16), _row_spec(_BC_ROWS, H),
         _full_spec((H, H)), _full_spec((1, H))],
        [_row_spec(_BC_ROWS, H), _full_spec((8, H))],
        [jax.ShapeDtypeStruct((NC, H), _F32),
         jax.ShapeDtypeStruct((8, H), _F32)],
    )(cm_sum, cnt_c, hc, w, b)


def _run_tk5(hc_pre, sums_s, g, b):
    return _tc_call1(
        _tk5_body, NC // _BC_ROWS,
        [_row_spec(_BC_ROWS, H), _full_spec((8, H)), _full_spec((1, H)),
         _full_spec((1, H))],
        [_row_spec(_BC_ROWS, H)],
        [jax.ShapeDtypeStruct((NC, H), _F32)],
    )(hc_pre, sums_s, g, b)


def _run_tk6(am_sum, cnt_a, h_mid, w, b):
    return _tc_call1(
        _tk6_body, N // _BN_ROWS,
        [_row_spec(_BN_ROWS, H), _row_spec(_BN_ROWS, 16), _row_spec(_BN_ROWS, H),
         _full_spec((H, H)), _full_spec((1, H))],
        [_row_spec(_BN_ROWS, H)],
        [jax.ShapeDtypeStruct((N, H), _F32)],
    )(am_sum, cnt_a, h_mid, w, b)


def kernel(x, x_clique, graph_lpe, edge_index_graph, edge_attr_graph,
           atom2clique_row, atom2clique_col,
           atom_emb, clique_emb, clique_W, clique_b, bond_emb, eps,
           W1, b1, bn1_g, bn1_b, W2, b2, gn_g, gn_b, sn_g, sn_b,
           a2c_W, a2c_b, c2a_W, c2a_b):
    i32 = jnp.int32
    f32 = jnp.float32

    x = x.astype(i32)
    t9 = atom_emb.reshape(9 * 100, H).astype(f32)
    idx9 = (x + 100 * jnp.arange(9, dtype=i32)[None, :]).T
    idx9 = jnp.pad(idx9, ((0, 0), (0, N_PAD - N))).reshape(9 * N_PAD)

    t4 = (clique_emb @ clique_W + clique_b).astype(f32)
    xc = jnp.pad(x_clique.astype(i32), (0, NC_PAD - NC))

    src = edge_index_graph[0].astype(i32)
    dst = edge_index_graph[1].astype(i32)
    ea = edge_attr_graph.astype(i32)
    ce = ea[:, 0] * 36 + ea[:, 1] * 6 + ea[:, 2]
    src_p = jnp.pad(src, (0, E_PAD - E))
    ce_p = jnp.pad(ce, (0, E_PAD - E))
    dst_p = jnp.pad(dst, (0, E_PAD - E), constant_values=-1)

    row = atom2clique_row.astype(i32)
    col = atom2clique_col.astype(i32)
    row_g = jnp.pad(row, (0, A_PAD - A))
    col_g = jnp.pad(col, (0, A_PAD - A))
    row_s = jnp.pad(row, (0, A_PAD - A), constant_values=-1)
    col_s = jnp.pad(col, (0, A_PAD - A), constant_values=-1)

    bts = [
        (bond_emb[l, 0][:, None, None, :] + bond_emb[l, 1][None, :, None, :]
         + bond_emb[l, 2][None, None, :, :]).reshape(216, H).astype(f32)
        for l in range(3)
    ]

    zeros_n = jnp.zeros((1568, H), f32)
    zeros_cnt = jnp.zeros((3128, 16), f32)
    ones16 = jnp.ones((CH, 16), f32)

    h0_pad, hc0_pad = _encode_sc(t9, idx9, t4, xc)
    h = h0_pad[:N]
    hc = hc0_pad[:NC]
    cnt_a, cnt_c = _counts_sc(row_s, col_s, ones16, zeros_cnt)

    for l in range(3):
        eps_l = jnp.full((8, 128), eps[l], f32)
        aggr = _edge_sc(h, bts[l], src_p, ce_p, dst_p, zeros_n)
        z1, sums1 = _run_tk1(h, aggr, eps_l, W1[l],
                             b1[l].reshape(1, 2 * H))
        z2, sums2 = _run_tk2(z1, sums1, bn1_g[l].reshape(1, 2 * H),
                             bn1_b[l].reshape(1, 2 * H), W2[l],
                             b2[l].reshape(1, H))
        h_mid = _run_tk3(z2, sums2, gn_g[l].reshape(1, H),
                         gn_b[l].reshape(1, H))
        cm_sum = _a2c_sc(h_mid, row_g, col_s, zeros_n)
        hc_pre, sums_s = _run_tk4(cm_sum, cnt_c, hc, a2c_W[l],
                                  a2c_b[l].reshape(1, H))
        hc = _run_tk5(hc_pre, sums_s, sn_g[l].reshape(1, H),
                      sn_b[l].reshape(1, H))
        am_sum = _c2a_sc(hc, col_g, row_s, zeros_n)
        h = _run_tk6(am_sum, cnt_a, h_mid, c2a_W[l],
                     c2a_b[l].reshape(1, H))

    return h


# speedup vs baseline: 6.1932x; 1.0477x over previous
import functools

import jax
import jax.numpy as jnp
from jax import lax
from jax.experimental import pallas as pl
from jax.experimental.pallas import tpu as pltpu
from jax.experimental.pallas import tpu_sc as plsc

N = 50000
NC = 25000
E = 800000
A = 50000
H = 64

NCORE = 2
NSUB = 16
CH = 128

E_PAD = 800768
CE = 128
A_PAD = 51200
N_PAD = 50048
NC_PAD = 25088

HALF_N = N // 2
HALF_NC = NC // 2
SP_N = 25088
SP_NC = 12544

_MESH = plsc.VectorSubcoreMesh(core_axis_name="c", subcore_axis_name="s")
_SC_PARAMS = pltpu.CompilerParams(use_tc_tiling_on_sc=False)


def _local_idx(dv, lo, half):
    valid = (dv >= lo) & (dv < lo + half)
    return jnp.where(valid, dv - lo, -1)


@functools.partial(
    pl.kernel,
    out_type=(
        jax.ShapeDtypeStruct((N_PAD, H), jnp.float32),
        jax.ShapeDtypeStruct((NC_PAD, H), jnp.float32),
    ),
    mesh=_MESH,
    compiler_params=_SC_PARAMS,
    scratch_types=[
        pltpu.VMEM((CH,), jnp.int32),
        pltpu.VMEM((CH, H), jnp.float32),
    ],
)
def _encode_sc(t9_hbm, idx9_hbm, t4_hbm, xc_hbm, h0_out, hc0_out,
               idxv, acc):
    w = lax.axis_index("s") * NCORE + lax.axis_index("c")

    n_chunks = N_PAD // CH

    @pl.loop(0, 13)
    def _(jj):
        j = w * 13 + jj

        @pl.when(j < n_chunks)
        def _():
            b = j * CH
            pltpu.sync_copy(idx9_hbm.at[pl.ds(b, CH)], idxv)
            pltpu.sync_copy(t9_hbm.at[idxv], acc)
            for i in range(1, 9):
                pltpu.sync_copy(idx9_hbm.at[pl.ds(i * N_PAD + b, CH)], idxv)
                pltpu.sync_copy(t9_hbm.at[idxv], acc, add=True)
            pltpu.sync_copy(acc, h0_out.at[pl.ds(b, CH), :])

    c_chunks = NC_PAD // CH

    @pl.loop(0, 7)
    def _(jj):
        j = w * 7 + jj

        @pl.when(j < c_chunks)
        def _():
            b = j * CH
            pltpu.sync_copy(xc_hbm.at[pl.ds(b, CH)], idxv)
            pltpu.sync_copy(t4_hbm.at[idxv], acc)
            pltpu.sync_copy(acc, hc0_out.at[pl.ds(b, CH), :])


@functools.partial(
    pl.kernel,
    out_type=(
        jax.ShapeDtypeStruct((N, 16), jnp.float32),
        jax.ShapeDtypeStruct((NC, 16), jnp.float32),
    ),
    mesh=_MESH,
    compiler_params=_SC_PARAMS,
    scratch_types=[
        pltpu.VMEM((CH,), jnp.int32),
        pltpu.VMEM((CH, 16), jnp.float32),
        pltpu.VMEM_SHARED((N_PAD, 16), jnp.float32),
    ],
)
def _counts_sc(row_s_hbm, col_s_hbm, ones_hbm, zc_hbm, cnt_a_out, cnt_c_out,
               idxv, ones_v, cnt_sh):
    c = lax.axis_index("c")
    s = lax.axis_index("s")
    pltpu.sync_copy(ones_hbm, ones_v)

    @pl.when(c == 0)
    def _():
        pltpu.sync_copy(zc_hbm, cnt_sh.at[pl.ds(s * 3128, 3128), :])

    @pl.when(c == 1)
    def _():
        pltpu.sync_copy(zc_hbm.at[pl.ds(0, 1564), :],
                        cnt_sh.at[pl.ds(s * 1564, 1564), :])

    plsc.subcore_barrier()

    n_chunks = A_PAD // (NSUB * CH)

    @pl.loop(0, n_chunks)
    def _(j):
        b = s * (A_PAD // NSUB) + j * CH

        @pl.when(c == 0)
        def _():
            pltpu.sync_copy(row_s_hbm.at[pl.ds(b, CH)], idxv)
            pltpu.sync_copy(
                ones_v, cnt_sh.at[plsc.Indices(idxv, ignored_value=-1)],
                add=True)

        @pl.when(c == 1)
        def _():
            pltpu.sync_copy(col_s_hbm.at[pl.ds(b, CH)], idxv)
            pltpu.sync_copy(
                ones_v, cnt_sh.at[plsc.Indices(idxv, ignored_value=-1)],
                add=True)

    plsc.subcore_barrier()

    @pl.when(c == 0)
    def _():
        @pl.when(s < 15)
        def _():
            pltpu.sync_copy(cnt_sh.at[pl.ds(s * 3128, 3128), :],
                            cnt_a_out.at[pl.ds(s * 3128, 3128), :])

        @pl.when(s == 15)
        def _():
            pltpu.sync_copy(cnt_sh.at[pl.ds(15 * 3128, 3080), :],
                            cnt_a_out.at[pl.ds(15 * 3128, 3080), :])

    @pl.when(c == 1)
    def _():
        @pl.when(s < 15)
        def _():
            pltpu.sync_copy(cnt_sh.at[pl.ds(s * 1564, 1564), :],
                            cnt_c_out.at[pl.ds(s * 1564, 1564), :])

        @pl.when(s == 15)
        def _():
            pltpu.sync_copy(cnt_sh.at[pl.ds(15 * 1564, 1540), :],
                            cnt_c_out.at[pl.ds(15 * 1564, 1540), :])


@functools.partial(
    pl.kernel,
    out_type=jax.ShapeDtypeStruct((N, H), jnp.float32),
    mesh=_MESH,
    compiler_params=_SC_PARAMS,
    scratch_types=[
        [pltpu.VMEM((CE,), jnp.int32)] * 2,
        [pltpu.VMEM((CE,), jnp.int32)] * 2,
        [pltpu.VMEM((CE,), jnp.int32)] * 2,
        [pltpu.VMEM((CE,), jnp.int32)] * 2,
        [pltpu.VMEM((CE, H), jnp.float32)] * 2,
        [pltpu.SemaphoreType.DMA] * 2,
        [pltpu.SemaphoreType.DMA] * 2,
        [pltpu.SemaphoreType.DMA] * 2,
        pltpu.VMEM_SHARED((SP_N, H), jnp.float32),
    ],
)
def _edge_sc(h_hbm, bt_hbm, src_hbm, ce_hbm, dst_hbm, z_hbm, aggr_out,
             srcv, cev, dstv, dlv, hbuf, sem_idx, sem_g, sem_s,
             aggr_sh):
    c = lax.axis_index("c")
    s = lax.axis_index("s")
    lo = c * HALF_N

    pltpu.sync_copy(z_hbm.at[pl.ds(0, 1568), :],
                    aggr_sh.at[pl.ds(s * 1568, 1568), :])
    plsc.subcore_barrier()

    per_sub = E_PAD // NSUB
    n_chunks = per_sub // CE

    def fire_idx(b, j):
        bb = s * per_sub + j * CE
        pltpu.async_copy(src_hbm.at[pl.ds(bb, CE)], srcv[b], sem_idx[b])
        pltpu.async_copy(ce_hbm.at[pl.ds(bb, CE)], cev[b], sem_idx[b])
        pltpu.async_copy(dst_hbm.at[pl.ds(bb, CE)], dstv[b], sem_idx[b])

    def wait_idx(b):
        pltpu.make_async_copy(src_hbm.at[pl.ds(0, CE)], srcv[b],
                              sem_idx[b]).wait()
        pltpu.make_async_copy(ce_hbm.at[pl.ds(0, CE)], cev[b],
                              sem_idx[b]).wait()
        pltpu.make_async_copy(dst_hbm.at[pl.ds(0, CE)], dstv[b],
                              sem_idx[b]).wait()

    def fire_h(b):
        pltpu.async_copy(h_hbm.at[srcv[b]], hbuf[b], sem_g[b])

    def wait_h(b):
        pltpu.make_async_copy(h_hbm.at[srcv[b]], hbuf[b], sem_g[b]).wait()

    def fire_bt_add(b):
        pltpu.async_copy(bt_hbm.at[cev[b]], hbuf[b], sem_g[b], add=True)

    def wait_bt(b):
        pltpu.make_async_copy(bt_hbm.at[cev[b]], hbuf[b], sem_g[b]).wait()

    def fire_scatter(b):
        @pl.loop(0, CE, unroll=8)
        def _(r):
            for q in range(H // 16):
                sl = pl.ds(q * 16, 16)
                hbuf[b][r, sl] = jnp.maximum(hbuf[b][r, sl], 0.0)

        for q in range(CE // 16):
            sl = pl.ds(q * 16, 16)
            dlv[b][sl] = _local_idx(dstv[b][sl], lo, HALF_N)
        pltpu.async_copy(
            hbuf[b], aggr_sh.at[plsc.Indices(dlv[b], ignored_value=-1)],
            sem_s[b], add=True)

    def wait_scatter(b):
        pltpu.make_async_copy(
            hbuf[b], aggr_sh.at[plsc.Indices(dlv[b], ignored_value=-1)],
            sem_s[b]).wait()

    fire_idx(0, 0)
    wait_idx(0)
    fire_h(0)
    fire_idx(1, 1)

    @pl.loop(0, n_chunks)
    def _(j):
        def body(b, nb):
            wait_h(b)
            fire_bt_add(b)

            @pl.when(j + 1 < n_chunks)
            def _():
                @pl.when(j >= 1)
                def _():
                    wait_scatter(nb)

                wait_idx(nb)
                fire_h(nb)

            wait_bt(b)
            fire_scatter(b)

            @pl.when(j + 2 < n_chunks)
            def _():
                fire_idx(b, j + 2)

        @pl.when(j % 2 == 0)
        def _():
            body(0, 1)

        @pl.when(j % 2 == 1)
        def _():
            body(1, 0)

    wait_scatter(0)
    wait_scatter(1)
    plsc.subcore_barrier()

    @pl.when(s < 15)
    def _():
        pltpu.sync_copy(aggr_sh.at[pl.ds(s * 1568, 1568), :],
                        aggr_out.at[pl.ds(lo + s * 1568, 1568), :])

    @pl.when(s == 15)
    def _():
        pltpu.sync_copy(aggr_sh.at[pl.ds(15 * 1568, 1480), :],
                        aggr_out.at[pl.ds(lo + 15 * 1568, 1480), :])


def _make_gss(table_rows, out_rows, sp_rows):
    half = out_rows // 2
    span = sp_rows // NSUB
    last = half - 15 * span
    n_chunks = A_PAD // (NSUB * CH)

    @functools.partial(
        pl.kernel,
        out_type=jax.ShapeDtypeStruct((out_rows, H), jnp.float32),
        mesh=_MESH,
        compiler_params=_SC_PARAMS,
        scratch_types=[
            pltpu.VMEM((CH,), jnp.int32),
            pltpu.VMEM((CH,), jnp.int32),
            pltpu.VMEM((CH,), jnp.int32),
            pltpu.VMEM((CH, H), jnp.float32),
            pltpu.VMEM_SHARED((sp_rows, H), jnp.float32),
        ],
    )
    def gss(table_hbm, gidx_hbm, sidx_hbm, z_hbm, out_hbm,
            gv, sv, dlv, buf, acc_sh):
        c = lax.axis_index("c")
        s = lax.axis_index("s")
        lo = c * half

        pltpu.sync_copy(z_hbm.at[pl.ds(0, span), :],
                        acc_sh.at[pl.ds(s * span, span), :])
        plsc.subcore_barrier()

        @pl.loop(0, n_chunks)
        def _(j):
            b = s * (A_PAD // NSUB) + j * CH
            pltpu.sync_copy(gidx_hbm.at[pl.ds(b, CH)], gv)
            pltpu.sync_copy(sidx_hbm.at[pl.ds(b, CH)], sv)
            pltpu.sync_copy(table_hbm.at[gv], buf)
            for k in range(CH // 16):
                sl = pl.ds(k * 16, 16)
                dlv[sl] = _local_idx(sv[sl], lo, half)
            pltpu.sync_copy(
                buf, acc_sh.at[plsc.Indices(dlv, ignored_value=-1)], add=True)

        plsc.subcore_barrier()

        @pl.when(s < 15)
        def _():
            pltpu.sync_copy(acc_sh.at[pl.ds(s * span, span), :],
                            out_hbm.at[pl.ds(lo + s * span, span), :])

        @pl.when(s == 15)
        def _():
            pltpu.sync_copy(acc_sh.at[pl.ds(15 * span, last), :],
                            out_hbm.at[pl.ds(lo + 15 * span, last), :])

    return gss


_a2c_sc = _make_gss(N, NC, SP_NC)
_c2a_sc = _make_gss(NC, N, SP_N)


_BN_EPS = 1e-5
_F32 = jnp.float32


def _dot(a, b):
    return jnp.dot(a, b, preferred_element_type=_F32)


def _stats_update(sums_ref, z, i):
    @pl.when(i == 0)
    def _():
        sums_ref[...] = jnp.zeros_like(sums_ref)

    sums_ref[0:1, :] += jnp.sum(z, axis=0, keepdims=True)
    sums_ref[1:2, :] += jnp.sum(z * z, axis=0, keepdims=True)


def _bn_apply(z, sums, nrows, g, b):
    m = sums[0:1, :] / nrows
    var = sums[1:2, :] / nrows - m * m
    return (z - m) * lax.rsqrt(var + _BN_EPS) * g + b


def _tk1_body(h_ref, aggr_ref, eps_ref, w1_ref, b1_ref, z1_ref, sums_ref):
    i = pl.program_id(0)
    u = (1.0 + eps_ref[0, 0]) * h_ref[...] + aggr_ref[...]
    z = _dot(u, w1_ref[...]) + b1_ref[...]
    z1_ref[...] = z
    _stats_update(sums_ref, z, i)


def _tk2_body(z1_ref, sums1_ref, g1_ref, bb1_ref, w2_ref, b2_ref,
              z2_ref, sums_ref):
    i = pl.program_id(0)
    v = jax.nn.relu(_bn_apply(z1_ref[...], sums1_ref[...], float(N),
                              g1_ref[...], bb1_ref[...]))
    z = _dot(v, w2_ref[...]) + b2_ref[...]
    z2_ref[...] = z
    _stats_update(sums_ref, z, i)


def _tk3_body(z2_ref, sums2_ref, g_ref, b_ref, h_ref):
    h_ref[...] = jax.nn.relu(_bn_apply(z2_ref[...], sums2_ref[...], float(N),
                                       g_ref[...], b_ref[...]))


def _tk4_body(cm_ref, cnt_ref, hc_ref, w_ref, b_ref, out_ref, sums_ref):
    i = pl.program_id(0)
    cm = cm_ref[...] / jnp.maximum(cnt_ref[:, 0:1], 1.0)
    z = hc_ref[...] + jax.nn.relu(_dot(cm, w_ref[...]) + b_ref[...])
    out_ref[...] = z
    _stats_update(sums_ref, z, i)


def _tk5_body(zp_ref, sums_ref, g_ref, b_ref, out_ref):
    out_ref[...] = jax.nn.relu(_bn_apply(zp_ref[...], sums_ref[...], float(NC),
                                         g_ref[...], b_ref[...]))


def _tk6_body(am_ref, cnt_ref, h_ref, w_ref, b_ref, out_ref):
    am = am_ref[...] / jnp.maximum(cnt_ref[:, 0:1], 1.0)
    out_ref[...] = h_ref[...] + jax.nn.relu(_dot(am, w_ref[...]) + b_ref[...])


def _row_spec(bs, cols):
    return pl.BlockSpec((bs, cols), lambda i: (i, 0))


def _full_spec(shape):
    return pl.BlockSpec(shape, lambda i: tuple(0 for _ in shape))


_BN_ROWS = 2000
_BC_ROWS = 1000


def _tc_call(body, grid, in_specs, out_specs, out_shapes):
    return pl.pallas_call(
        body, grid=(grid,), in_specs=in_specs, out_specs=out_specs,
        out_shape=out_shapes)


def _tc_call1(*args):
    def run(*ins):
        (out,) = _tc_call(*args)(*ins)
        return out
    return run


def _run_tk1(h, aggr, eps_l, w1, b1):
    return _tc_call(
        _tk1_body, N // _BN_ROWS,
        [_row_spec(_BN_ROWS, H), _row_spec(_BN_ROWS, H), _full_spec((8, 128)),
         _full_spec((H, 2 * H)), _full_spec((1, 2 * H))],
        [_row_spec(_BN_ROWS, 2 * H), _full_spec((8, 2 * H))],
        [jax.ShapeDtypeStruct((N, 2 * H), _F32),
         jax.ShapeDtypeStruct((8, 2 * H), _F32)],
    )(h, aggr, eps_l, w1, b1)


def _run_tk2(z1, sums1, g1, bb1, w2, b2):
    return _tc_call(
        _tk2_body, N // _BN_ROWS,
        [_row_spec(_BN_ROWS, 2 * H), _full_spec((8, 2 * H)),
         _full_spec((1, 2 * H)), _full_spec((1, 2 * H)),
         _full_spec((2 * H, H)), _full_spec((1, H))],
        [_row_spec(_BN_ROWS, H), _full_spec((8, H))],
        [jax.ShapeDtypeStruct((N, H), _F32),
         jax.ShapeDtypeStruct((8, H), _F32)],
    )(z1, sums1, g1, bb1, w2, b2)


def _run_tk3(z2, sums2, g, b):
    return _tc_call1(
        _tk3_body, N // _BN_ROWS,
        [_row_spec(_BN_ROWS, H), _full_spec((8, H)), _full_spec((1, H)),
         _full_spec((1, H))],
        [_row_spec(_BN_ROWS, H)],
        [jax.ShapeDtypeStruct((N, H), _F32)],
    )(z2, sums2, g, b)


def _run_tk4(cm_sum, cnt_c, hc, w, b):
    return _tc_call(
        _tk4_body, NC // _BC_ROWS,
        [_row_spec(_BC_ROWS, H), _row_spec(_BC_ROWS, 16), _row_spec(_BC_ROWS, H),
         _full_spec((H, H)), _full_spec((1, H))],
        [_row_spec(_BC_ROWS, H), _full_spec((8, H))],
        [jax.ShapeDtypeStruct((NC, H), _F32),
         jax.ShapeDtypeStruct((8, H), _F32)],
    )(cm_sum, cnt_c, hc, w, b)


def _run_tk5(hc_pre, sums_s, g, b):
    return _tc_call1(
        _tk5_body, NC // _BC_ROWS,
        [_row_spec(_BC_ROWS, H), _full_spec((8, H)), _full_spec((1, H)),
         _full_spec((1, H))],
        [_row_spec(_BC_ROWS, H)],
        [jax.ShapeDtypeStruct((NC, H), _F32)],
    )(hc_pre, sums_s, g, b)


def _run_tk6(am_sum, cnt_a, h_mid, w, b):
    return _tc_call1(
        _tk6_body, N // _BN_ROWS,
        [_row_spec(_BN_ROWS, H), _row_spec(_BN_ROWS, 16), _row_spec(_BN_ROWS, H),
         _full_spec((H, H)), _full_spec((1, H))],
        [_row_spec(_BN_ROWS, H)],
        [jax.ShapeDtypeStruct((N, H), _F32)],
    )(am_sum, cnt_a, h_mid, w, b)


def kernel(x, x_clique, graph_lpe, edge_index_graph, edge_attr_graph,
           atom2clique_row, atom2clique_col,
           atom_emb, clique_emb, clique_W, clique_b, bond_emb, eps,
           W1, b1, bn1_g, bn1_b, W2, b2, gn_g, gn_b, sn_g, sn_b,
           a2c_W, a2c_b, c2a_W, c2a_b):
    i32 = jnp.int32
    f32 = jnp.float32

    x = x.astype(i32)
    t9 = atom_emb.reshape(9 * 100, H).astype(f32)
    idx9 = (x + 100 * jnp.arange(9, dtype=i32)[None, :]).T
    idx9 = jnp.pad(idx9, ((0, 0), (0, N_PAD - N))).reshape(9 * N_PAD)

    t4 = (clique_emb @ clique_W + clique_b).astype(f32)
    xc = jnp.pad(x_clique.astype(i32), (0, NC_PAD - NC))

    src = edge_index_graph[0].astype(i32)
    dst = edge_index_graph[1].astype(i32)
    ea = edge_attr_graph.astype(i32)
    ce = ea[:, 0] * 36 + ea[:, 1] * 6 + ea[:, 2]
    src_p = jnp.pad(src, (0, E_PAD - E))
    ce_p = jnp.pad(ce, (0, E_PAD - E))
    dst_p = jnp.pad(dst, (0, E_PAD - E), constant_values=-1)

    row = atom2clique_row.astype(i32)
    col = atom2clique_col.astype(i32)
    row_g = jnp.pad(row, (0, A_PAD - A))
    col_g = jnp.pad(col, (0, A_PAD - A))
    row_s = jnp.pad(row, (0, A_PAD - A), constant_values=-1)
    col_s = jnp.pad(col, (0, A_PAD - A), constant_values=-1)

    bts = [
        (bond_emb[l, 0][:, None, None, :] + bond_emb[l, 1][None, :, None, :]
         + bond_emb[l, 2][None, None, :, :]).reshape(216, H).astype(f32)
        for l in range(3)
    ]

    zeros_n = jnp.zeros((1568, H), f32)
    zeros_cnt = jnp.zeros((3128, 16), f32)
    ones16 = jnp.ones((CH, 16), f32)

    h0_pad, hc0_pad = _encode_sc(t9, idx9, t4, xc)
    h = h0_pad[:N]
    hc = hc0_pad[:NC]
    cnt_a, cnt_c = _counts_sc(row_s, col_s, ones16, zeros_cnt)

    for l in range(3):
        eps_l = jnp.full((8, 128), eps[l], f32)
        aggr = _edge_sc(h, bts[l], src_p, ce_p, dst_p, zeros_n)
        z1, sums1 = _run_tk1(h, aggr, eps_l, W1[l],
                             b1[l].reshape(1, 2 * H))
        z2, sums2 = _run_tk2(z1, sums1, bn1_g[l].reshape(1, 2 * H),
                             bn1_b[l].reshape(1, 2 * H), W2[l],
                             b2[l].reshape(1, H))
        h_mid = _run_tk3(z2, sums2, gn_g[l].reshape(1, H),
                         gn_b[l].reshape(1, H))
        cm_sum = _a2c_sc(h_mid, row_g, col_s, zeros_n)
        hc_pre, sums_s = _run_tk4(cm_sum, cnt_c, hc, a2c_W[l],
                                  a2c_b[l].reshape(1, H))
        hc = _run_tk5(hc_pre, sums_s, sn_g[l].reshape(1, H),
                      sn_b[l].reshape(1, H))
        am_sum = _c2a_sc(hc, col_g, row_s, zeros_n)
        h = _run_tk6(am_sum, cnt_a, h_mid, c2a_W[l],
                     c2a_b[l].reshape(1, H))

    return h
